# Initial kernel scaffold; baseline (speedup 1.0000x reference)
import jax
import jax.numpy as jnp
from jax.experimental import pallas as pl


def kernel(x, edge_index, batch, W1_0, b1_0, W2_0, b2_0, gam_0, bet_0, W1_1, b1_1, W2_1, b2_1, gam_1, bet_1, W1_2, b1_2, W2_2, b2_2, gam_2, bet_2, Wm1, bm1, Wm2, bm2):
    raise NotImplementedError("write your pallas kernel here")


# speedup vs baseline: 5.8019x; 5.8019x over previous
import functools

import jax
import jax.numpy as jnp
from jax import lax
from jax.experimental import pallas as pl
from jax.experimental.pallas import tpu as pltpu
from jax.experimental.pallas import tpu_sc as plsc

_N = 10000
_E = 320000
_G = 64
_NC = 2
_NS = 16
_CH = 125
_RPT = 624
_RPT_LAST = _N - (_NS - 1) * _RPT
_GRP = 16

_BR = 2000
_NSTEPS = _N // _BR


def _make_sc_agg(split_edges: bool):
    if split_edges:
        nch = (_E // _NC) // _NS // _CH
        core_row_off = (_E // _NC) // _CH
    else:
        nch = _E // _NS // _CH
        core_row_off = 0
    ngrp = nch // _GRP

    def body(h_a, h_b, src_r, dst_r, zblk, out_a, out_b,
             acc, idxs, idxd, rows, sem):
        c = lax.axis_index("c")
        s = lax.axis_index("s")

        @pl.when(s < _NS - 1)
        def _():
            pltpu.sync_copy(zblk.at[pl.ds(0, _RPT)],
                            acc.at[pl.ds(s * _RPT, _RPT)])

        @pl.when(s == _NS - 1)
        def _():
            pltpu.sync_copy(zblk, acc.at[pl.ds((_NS - 1) * _RPT, _RPT_LAST)])

        row0 = c * core_row_off + s * nch
        plsc.subcore_barrier()

        def chunk(h):
            def group(g, carry):
                gr = row0 + g * _GRP
                pltpu.sync_copy(src_r.at[pl.ds(gr, _GRP)], idxs)
                pltpu.sync_copy(dst_r.at[pl.ds(gr, _GRP)], idxd)

                def step(j, carry2):
                    cp = pltpu.make_async_copy(h.at[idxs.at[j]], rows, sem)
                    cp.start()
                    cp.wait()
                    pltpu.sync_copy(rows, acc.at[idxd.at[j]], add=True)
                    return carry2
                lax.fori_loop(0, _GRP, step, 0)
                return carry
            lax.fori_loop(0, ngrp, group, 0)

        @pl.when(c == 0)
        def _():
            chunk(h_a)

        @pl.when(c == 1)
        def _():
            chunk(h_b)

        plsc.subcore_barrier()

        def copy_out(out):
            @pl.when(s < _NS - 1)
            def _():
                pltpu.sync_copy(acc.at[pl.ds(s * _RPT, _RPT)],
                                out.at[pl.ds(s * _RPT, _RPT)])

            @pl.when(s == _NS - 1)
            def _():
                pltpu.sync_copy(acc.at[pl.ds((_NS - 1) * _RPT, _RPT_LAST)],
                                out.at[pl.ds((_NS - 1) * _RPT, _RPT_LAST)])

        @pl.when(c == 0)
        def _():
            copy_out(out_a)

        @pl.when(c == 1)
        def _():
            copy_out(out_b)

    mesh = plsc.VectorSubcoreMesh(core_axis_name="c", subcore_axis_name="s")
    return pl.kernel(
        body,
        out_type=(jax.ShapeDtypeStruct((_N, 128), jnp.float32),
                  jax.ShapeDtypeStruct((_N, 128), jnp.float32)),
        mesh=mesh,
        scratch_types=[
            pltpu.VMEM_SHARED((_N, 128), jnp.float32),
            pltpu.VMEM((_GRP, _CH), jnp.int32),
            pltpu.VMEM((_GRP, _CH), jnp.int32),
            pltpu.VMEM((_CH, 128), jnp.float32),
            pltpu.SemaphoreType.DMA,
        ],
    )


def _pre0_body(x_ref, p0_ref, p1_ref, w1_ref, b1_ref, w2_ref, b2_ref,
               zpre_ref, stats_ref):
    i = pl.program_id(0)
    zin = x_ref[...] + p0_ref[...] + p1_ref[...]
    z1 = jnp.maximum(jnp.dot(zin, w1_ref[...]) + b1_ref[...], 0.0)
    zp = jnp.dot(z1, w2_ref[...]) + b2_ref[...]
    zpre_ref[...] = zp
    st = jnp.concatenate(
        [jnp.sum(zp, axis=0, keepdims=True),
         jnp.sum(zp * zp, axis=0, keepdims=True),
         jnp.zeros((6, zp.shape[1]), jnp.float32)], axis=0)

    @pl.when(i == 0)
    def _():
        stats_ref[...] = st

    @pl.when(i != 0)
    def _():
        stats_ref[...] = stats_ref[...] + st


def _pre12_body(h0_ref, h1_ref, a0_ref, a1_ref, w1_ref, b1_ref, w2_ref,
                b2_ref, zpre_ref, stats_ref):
    i = pl.program_id(0)
    zin0 = h0_ref[...] + a0_ref[...]
    zin1 = h1_ref[...] + a1_ref[...]
    z1 = jnp.maximum(
        jnp.dot(zin0, w1_ref[0:128, :]) + jnp.dot(zin1, w1_ref[128:256, :])
        + b1_ref[...], 0.0)
    zp = jnp.dot(z1, w2_ref[...]) + b2_ref[...]
    zpre_ref[...] = zp
    st = jnp.concatenate(
        [jnp.sum(zp, axis=0, keepdims=True),
         jnp.sum(zp * zp, axis=0, keepdims=True),
         jnp.zeros((6, zp.shape[1]), jnp.float32)], axis=0)

    @pl.when(i == 0)
    def _():
        stats_ref[...] = st

    @pl.when(i != 0)
    def _():
        stats_ref[...] = stats_ref[...] + st


def _row_block(din):
    return pl.BlockSpec((_BR, din), lambda i: (i, 0))


def _full_block(shape):
    return pl.BlockSpec(shape, lambda i: tuple(0 for _ in shape))


def _tc_pre0(x, p0, p1, w1, b1, w2, b2):
    return pl.pallas_call(
        _pre0_body,
        grid=(_NSTEPS,),
        in_specs=[_row_block(128), _row_block(128), _row_block(128),
                  _full_block((128, 256)), _full_block((1, 256)),
                  _full_block((256, 256)), _full_block((1, 256))],
        out_specs=[pl.BlockSpec((_BR, 256), lambda i: (i, 0)),
                   _full_block((8, 256))],
        out_shape=[jax.ShapeDtypeStruct((_N, 256), jnp.float32),
                   jax.ShapeDtypeStruct((8, 256), jnp.float32)],
    )(x, p0, p1, w1, b1, w2, b2)


def _tc_pre12(h0, h1, a0, a1, w1, b1, w2, b2):
    return pl.pallas_call(
        _pre12_body,
        grid=(_NSTEPS,),
        in_specs=[_row_block(128), _row_block(128), _row_block(128),
                  _row_block(128),
                  _full_block((256, 256)), _full_block((1, 256)),
                  _full_block((256, 256)), _full_block((1, 256))],
        out_specs=[pl.BlockSpec((_BR, 256), lambda i: (i, 0)),
                   _full_block((8, 256))],
        out_shape=[jax.ShapeDtypeStruct((_N, 256), jnp.float32),
                   jax.ShapeDtypeStruct((8, 256), jnp.float32)],
    )(h0, h1, a0, a1, w1, b1, w2, b2)


def _post_body_res(zpre_ref, stats_ref, gam_ref, bet_ref, h0_ref, h1_ref,
                   o0_ref, o1_ref):
    _post_common(zpre_ref, stats_ref, gam_ref, bet_ref, h0_ref, h1_ref,
                 o0_ref, o1_ref)


def _post_body_nores(zpre_ref, stats_ref, gam_ref, bet_ref, o0_ref, o1_ref):
    _post_common(zpre_ref, stats_ref, gam_ref, bet_ref, None, None,
                 o0_ref, o1_ref)


def _post_common(zpre_ref, stats_ref, gam_ref, bet_ref, h0_ref, h1_ref,
                 o0_ref, o1_ref):
    inv_n = 1.0 / _N
    mean = stats_ref[0:1, :] * inv_n
    ex2 = stats_ref[1:2, :] * inv_n
    var = ex2 - mean * mean
    scale = gam_ref[...] / jnp.sqrt(var + 1e-5)
    y = (zpre_ref[...] - mean) * scale + bet_ref[...]
    y = jnp.maximum(y, 0.0)
    y0 = y[:, 0:128]
    y1 = y[:, 128:256]
    if h0_ref is not None:
        y0 = y0 + h0_ref[...]
        y1 = y1 + h1_ref[...]
    o0_ref[...] = y0
    o1_ref[...] = y1


def _tc_post(zpre, stats, gam, bet, h0=None, h1=None):
    residual = h0 is not None
    in_specs = [pl.BlockSpec((_BR, 256), lambda i: (i, 0)),
                _full_block((8, 256)), _full_block((1, 256)),
                _full_block((1, 256))]
    args = [zpre, stats, gam, bet]
    if residual:
        in_specs += [_row_block(128), _row_block(128)]
        args += [h0, h1]
    return pl.pallas_call(
        _post_body_res if residual else _post_body_nores,
        grid=(_NSTEPS,),
        in_specs=in_specs,
        out_specs=[_row_block(128), _row_block(128)],
        out_shape=[jax.ShapeDtypeStruct((_N, 128), jnp.float32),
                   jax.ShapeDtypeStruct((_N, 128), jnp.float32)],
    )(*args)


def _pool_body(h0_ref, h1_ref, batch_ref, wm1_ref, bm1_ref, wm2_ref, bm2_ref,
               out_ref, sums, counts):
    i = pl.program_id(0)
    b = batch_ref[0]
    gids = lax.broadcasted_iota(jnp.int32, (_G, _BR), 0)
    mask = (gids == b).astype(jnp.float32)
    hcat = jnp.concatenate([h0_ref[...], h1_ref[...]], axis=1)
    part = jnp.dot(mask, hcat)
    cnt = jnp.broadcast_to(jnp.sum(mask, axis=1, keepdims=True), (_G, 128))

    @pl.when(i == 0)
    def _():
        sums[...] = part
        counts[...] = cnt

    @pl.when(i != 0)
    def _():
        sums[...] = sums[...] + part
        counts[...] = counts[...] + cnt

    @pl.when(i == _NSTEPS - 1)
    def _():
        hg = sums[...] / jnp.maximum(counts[:, 0:1], 1.0)
        t = jnp.maximum(jnp.dot(hg, wm1_ref[...]) + bm1_ref[...], 0.0)
        out_ref[...] = jnp.dot(t, wm2_ref[...]) + bm2_ref[...]


def _tc_pool_head(h0, h1, batch3, wm1, bm1, wm2p, bm2p):
    return pl.pallas_call(
        _pool_body,
        grid=(_NSTEPS,),
        in_specs=[_row_block(128), _row_block(128),
                  pl.BlockSpec((1, 1, _BR), lambda i: (i, 0, 0)),
                  _full_block((256, 256)), _full_block((1, 256)),
                  _full_block((256, 128)), _full_block((1, 128))],
        out_specs=_full_block((_G, 128)),
        out_shape=jax.ShapeDtypeStruct((_G, 128), jnp.float32),
        scratch_shapes=[pltpu.VMEM((_G, 256), jnp.float32),
                        pltpu.VMEM((_G, 128), jnp.float32)],
    )(h0, h1, batch3, wm1, bm1, wm2p, bm2p)


def kernel(x, edge_index, batch, W1_0, b1_0, W2_0, b2_0, gam_0, bet_0,
           W1_1, b1_1, W2_1, b2_1, gam_1, bet_1,
           W1_2, b1_2, W2_2, b2_2, gam_2, bet_2, Wm1, bm1, Wm2, bm2):
    src_r = edge_index[0].astype(jnp.int32).reshape(_E // _CH, _CH)
    dst_r = edge_index[1].astype(jnp.int32).reshape(_E // _CH, _CH)
    zblk = jnp.zeros((_RPT_LAST, 128), jnp.float32)

    r = lambda v: v.reshape(1, -1)
    agg_edges = _make_sc_agg(split_edges=True)
    agg_feats = _make_sc_agg(split_edges=False)

    p0, p1 = agg_edges(x, x, src_r, dst_r, zblk)
    zpre, stats = _tc_pre0(x, p0, p1, W1_0, r(b1_0), W2_0, r(b2_0))
    h0, h1 = _tc_post(zpre, stats, r(gam_0), r(bet_0))

    for (W1, b1, W2, b2, gam, bet) in (
            (W1_1, b1_1, W2_1, b2_1, gam_1, bet_1),
            (W1_2, b1_2, W2_2, b2_2, gam_2, bet_2)):
        a0, a1 = agg_feats(h0, h1, src_r, dst_r, zblk)
        zpre, stats = _tc_pre12(h0, h1, a0, a1, W1, r(b1), W2, r(b2))
        h0, h1 = _tc_post(zpre, stats, r(gam), r(bet), h0, h1)

    batch3 = batch.astype(jnp.int32).reshape(_NSTEPS, 1, _BR)
    wm2p = jnp.pad(Wm2, ((0, 0), (0, 118)))
    bm2p = jnp.pad(bm2, (0, 118)).reshape(1, 128)
    out = _tc_pool_head(h0, h1, batch3, Wm1, r(bm1), wm2p, bm2p)
    return out[:, :10]


# speedup vs baseline: 8.4779x; 1.4612x over previous
import functools

import jax
import jax.numpy as jnp
from jax import lax
from jax.experimental import pallas as pl
from jax.experimental.pallas import tpu as pltpu
from jax.experimental.pallas import tpu_sc as plsc

_N = 10000
_E = 320000
_G = 64
_NC = 2
_NS = 16
_CH = 125
_RPT = 624
_RPT_LAST = _N - (_NS - 1) * _RPT
_GRP = 16

_BR = 2000
_NSTEPS = _N // _BR


def _make_sc_agg(split_edges: bool):
    if split_edges:
        nch = (_E // _NC) // _NS // _CH
        core_row_off = (_E // _NC) // _CH
    else:
        nch = _E // _NS // _CH
        core_row_off = 0
    ngrp = nch // _GRP

    def body(h_a, h_b, src_r, dst_r, zblk, out_a, out_b,
             acc, idxs, idxd, rows0, rows1, sem0, sem1):
        c = lax.axis_index("c")
        s = lax.axis_index("s")

        @pl.when(s < _NS - 1)
        def _():
            pltpu.sync_copy(zblk.at[pl.ds(0, _RPT)],
                            acc.at[pl.ds(s * _RPT, _RPT)])

        @pl.when(s == _NS - 1)
        def _():
            pltpu.sync_copy(zblk, acc.at[pl.ds((_NS - 1) * _RPT, _RPT_LAST)])

        row0 = c * core_row_off + s * nch
        plsc.subcore_barrier()

        def chunk(h):
            rbuf = (rows0, rows1)
            sems = (sem0, sem1)

            def group(g, carry):
                gr = row0 + g * _GRP
                pltpu.sync_copy(src_r.at[pl.ds(gr, _GRP)], idxs)
                pltpu.sync_copy(dst_r.at[pl.ds(gr, _GRP)], idxd)
                pltpu.make_async_copy(h.at[idxs.at[0]], rbuf[0],
                                      sems[0]).start()
                for j in range(_GRP):
                    if j + 1 < _GRP:
                        pltpu.make_async_copy(h.at[idxs.at[j + 1]],
                                              rbuf[(j + 1) % 2],
                                              sems[(j + 1) % 2]).start()
                    pltpu.make_async_copy(h.at[idxs.at[j]], rbuf[j % 2],
                                          sems[j % 2]).wait()
                    pltpu.sync_copy(rbuf[j % 2], acc.at[idxd.at[j]], add=True)
                return carry
            lax.fori_loop(0, ngrp, group, 0)

        @pl.when(c == 0)
        def _():
            chunk(h_a)

        @pl.when(c == 1)
        def _():
            chunk(h_b)

        plsc.subcore_barrier()

        def copy_out(out):
            @pl.when(s < _NS - 1)
            def _():
                pltpu.sync_copy(acc.at[pl.ds(s * _RPT, _RPT)],
                                out.at[pl.ds(s * _RPT, _RPT)])

            @pl.when(s == _NS - 1)
            def _():
                pltpu.sync_copy(acc.at[pl.ds((_NS - 1) * _RPT, _RPT_LAST)],
                                out.at[pl.ds((_NS - 1) * _RPT, _RPT_LAST)])

        @pl.when(c == 0)
        def _():
            copy_out(out_a)

        @pl.when(c == 1)
        def _():
            copy_out(out_b)

    mesh = plsc.VectorSubcoreMesh(core_axis_name="c", subcore_axis_name="s")
    return pl.kernel(
        body,
        out_type=(jax.ShapeDtypeStruct((_N, 128), jnp.float32),
                  jax.ShapeDtypeStruct((_N, 128), jnp.float32)),
        mesh=mesh,
        scratch_types=[
            pltpu.VMEM_SHARED((_N, 128), jnp.float32),
            pltpu.VMEM((_GRP, _CH), jnp.int32),
            pltpu.VMEM((_GRP, _CH), jnp.int32),
            pltpu.VMEM((_CH, 128), jnp.float32),
            pltpu.VMEM((_CH, 128), jnp.float32),
            pltpu.SemaphoreType.DMA,
            pltpu.SemaphoreType.DMA,
        ],
    )


def _pre0_body(x_ref, p0_ref, p1_ref, w1_ref, b1_ref, w2_ref, b2_ref,
               zpre_ref, stats_ref):
    i = pl.program_id(0)
    zin = x_ref[...] + p0_ref[...] + p1_ref[...]
    z1 = jnp.maximum(jnp.dot(zin, w1_ref[...]) + b1_ref[...], 0.0)
    zp = jnp.dot(z1, w2_ref[...]) + b2_ref[...]
    zpre_ref[...] = zp
    st = jnp.concatenate(
        [jnp.sum(zp, axis=0, keepdims=True),
         jnp.sum(zp * zp, axis=0, keepdims=True),
         jnp.zeros((6, zp.shape[1]), jnp.float32)], axis=0)

    @pl.when(i == 0)
    def _():
        stats_ref[...] = st

    @pl.when(i != 0)
    def _():
        stats_ref[...] = stats_ref[...] + st


def _pre12_body(h0_ref, h1_ref, a0_ref, a1_ref, w1_ref, b1_ref, w2_ref,
                b2_ref, zpre_ref, stats_ref):
    i = pl.program_id(0)
    zin0 = h0_ref[...] + a0_ref[...]
    zin1 = h1_ref[...] + a1_ref[...]
    z1 = jnp.maximum(
        jnp.dot(zin0, w1_ref[0:128, :]) + jnp.dot(zin1, w1_ref[128:256, :])
        + b1_ref[...], 0.0)
    zp = jnp.dot(z1, w2_ref[...]) + b2_ref[...]
    zpre_ref[...] = zp
    st = jnp.concatenate(
        [jnp.sum(zp, axis=0, keepdims=True),
         jnp.sum(zp * zp, axis=0, keepdims=True),
         jnp.zeros((6, zp.shape[1]), jnp.float32)], axis=0)

    @pl.when(i == 0)
    def _():
        stats_ref[...] = st

    @pl.when(i != 0)
    def _():
        stats_ref[...] = stats_ref[...] + st


def _row_block(din):
    return pl.BlockSpec((_BR, din), lambda i: (i, 0))


def _full_block(shape):
    return pl.BlockSpec(shape, lambda i: tuple(0 for _ in shape))


def _tc_pre0(x, p0, p1, w1, b1, w2, b2):
    return pl.pallas_call(
        _pre0_body,
        grid=(_NSTEPS,),
        in_specs=[_row_block(128), _row_block(128), _row_block(128),
                  _full_block((128, 256)), _full_block((1, 256)),
                  _full_block((256, 256)), _full_block((1, 256))],
        out_specs=[pl.BlockSpec((_BR, 256), lambda i: (i, 0)),
                   _full_block((8, 256))],
        out_shape=[jax.ShapeDtypeStruct((_N, 256), jnp.float32),
                   jax.ShapeDtypeStruct((8, 256), jnp.float32)],
    )(x, p0, p1, w1, b1, w2, b2)


def _tc_pre12(h0, h1, a0, a1, w1, b1, w2, b2):
    return pl.pallas_call(
        _pre12_body,
        grid=(_NSTEPS,),
        in_specs=[_row_block(128), _row_block(128), _row_block(128),
                  _row_block(128),
                  _full_block((256, 256)), _full_block((1, 256)),
                  _full_block((256, 256)), _full_block((1, 256))],
        out_specs=[pl.BlockSpec((_BR, 256), lambda i: (i, 0)),
                   _full_block((8, 256))],
        out_shape=[jax.ShapeDtypeStruct((_N, 256), jnp.float32),
                   jax.ShapeDtypeStruct((8, 256), jnp.float32)],
    )(h0, h1, a0, a1, w1, b1, w2, b2)


def _post_body_res(zpre_ref, stats_ref, gam_ref, bet_ref, h0_ref, h1_ref,
                   o0_ref, o1_ref):
    _post_common(zpre_ref, stats_ref, gam_ref, bet_ref, h0_ref, h1_ref,
                 o0_ref, o1_ref)


def _post_body_nores(zpre_ref, stats_ref, gam_ref, bet_ref, o0_ref, o1_ref):
    _post_common(zpre_ref, stats_ref, gam_ref, bet_ref, None, None,
                 o0_ref, o1_ref)


def _post_common(zpre_ref, stats_ref, gam_ref, bet_ref, h0_ref, h1_ref,
                 o0_ref, o1_ref):
    inv_n = 1.0 / _N
    mean = stats_ref[0:1, :] * inv_n
    ex2 = stats_ref[1:2, :] * inv_n
    var = ex2 - mean * mean
    scale = gam_ref[...] / jnp.sqrt(var + 1e-5)
    y = (zpre_ref[...] - mean) * scale + bet_ref[...]
    y = jnp.maximum(y, 0.0)
    y0 = y[:, 0:128]
    y1 = y[:, 128:256]
    if h0_ref is not None:
        y0 = y0 + h0_ref[...]
        y1 = y1 + h1_ref[...]
    o0_ref[...] = y0
    o1_ref[...] = y1


def _tc_post(zpre, stats, gam, bet, h0=None, h1=None):
    residual = h0 is not None
    in_specs = [pl.BlockSpec((_BR, 256), lambda i: (i, 0)),
                _full_block((8, 256)), _full_block((1, 256)),
                _full_block((1, 256))]
    args = [zpre, stats, gam, bet]
    if residual:
        in_specs += [_row_block(128), _row_block(128)]
        args += [h0, h1]
    return pl.pallas_call(
        _post_body_res if residual else _post_body_nores,
        grid=(_NSTEPS,),
        in_specs=in_specs,
        out_specs=[_row_block(128), _row_block(128)],
        out_shape=[jax.ShapeDtypeStruct((_N, 128), jnp.float32),
                   jax.ShapeDtypeStruct((_N, 128), jnp.float32)],
    )(*args)


def _pool_body(h0_ref, h1_ref, batch_ref, wm1_ref, bm1_ref, wm2_ref, bm2_ref,
               out_ref, sums, counts):
    i = pl.program_id(0)
    b = batch_ref[0]
    gids = lax.broadcasted_iota(jnp.int32, (_G, _BR), 0)
    mask = (gids == b).astype(jnp.float32)
    hcat = jnp.concatenate([h0_ref[...], h1_ref[...]], axis=1)
    part = jnp.dot(mask, hcat)
    cnt = jnp.broadcast_to(jnp.sum(mask, axis=1, keepdims=True), (_G, 128))

    @pl.when(i == 0)
    def _():
        sums[...] = part
        counts[...] = cnt

    @pl.when(i != 0)
    def _():
        sums[...] = sums[...] + part
        counts[...] = counts[...] + cnt

    @pl.when(i == _NSTEPS - 1)
    def _():
        hg = sums[...] / jnp.maximum(counts[:, 0:1], 1.0)
        t = jnp.maximum(jnp.dot(hg, wm1_ref[...]) + bm1_ref[...], 0.0)
        out_ref[...] = jnp.dot(t, wm2_ref[...]) + bm2_ref[...]


def _tc_pool_head(h0, h1, batch3, wm1, bm1, wm2p, bm2p):
    return pl.pallas_call(
        _pool_body,
        grid=(_NSTEPS,),
        in_specs=[_row_block(128), _row_block(128),
                  pl.BlockSpec((1, 1, _BR), lambda i: (i, 0, 0)),
                  _full_block((256, 256)), _full_block((1, 256)),
                  _full_block((256, 128)), _full_block((1, 128))],
        out_specs=_full_block((_G, 128)),
        out_shape=jax.ShapeDtypeStruct((_G, 128), jnp.float32),
        scratch_shapes=[pltpu.VMEM((_G, 256), jnp.float32),
                        pltpu.VMEM((_G, 128), jnp.float32)],
    )(h0, h1, batch3, wm1, bm1, wm2p, bm2p)


def kernel(x, edge_index, batch, W1_0, b1_0, W2_0, b2_0, gam_0, bet_0,
           W1_1, b1_1, W2_1, b2_1, gam_1, bet_1,
           W1_2, b1_2, W2_2, b2_2, gam_2, bet_2, Wm1, bm1, Wm2, bm2):
    src_r = edge_index[0].astype(jnp.int32).reshape(_E // _CH, _CH)
    dst_r = edge_index[1].astype(jnp.int32).reshape(_E // _CH, _CH)
    zblk = jnp.zeros((_RPT_LAST, 128), jnp.float32)

    r = lambda v: v.reshape(1, -1)
    agg_edges = _make_sc_agg(split_edges=True)
    agg_feats = _make_sc_agg(split_edges=False)

    p0, p1 = agg_edges(x, x, src_r, dst_r, zblk)
    zpre, stats = _tc_pre0(x, p0, p1, W1_0, r(b1_0), W2_0, r(b2_0))
    h0, h1 = _tc_post(zpre, stats, r(gam_0), r(bet_0))

    for (W1, b1, W2, b2, gam, bet) in (
            (W1_1, b1_1, W2_1, b2_1, gam_1, bet_1),
            (W1_2, b1_2, W2_2, b2_2, gam_2, bet_2)):
        a0, a1 = agg_feats(h0, h1, src_r, dst_r, zblk)
        zpre, stats = _tc_pre12(h0, h1, a0, a1, W1, r(b1), W2, r(b2))
        h0, h1 = _tc_post(zpre, stats, r(gam), r(bet), h0, h1)

    batch3 = batch.astype(jnp.int32).reshape(_NSTEPS, 1, _BR)
    wm2p = jnp.pad(Wm2, ((0, 0), (0, 118)))
    bm2p = jnp.pad(bm2, (0, 118)).reshape(1, 128)
    out = _tc_pool_head(h0, h1, batch3, Wm1, r(bm1), wm2p, bm2p)
    return out[:, :10]


# speedup vs baseline: 8.6352x; 1.0186x over previous
import functools

import jax
import jax.numpy as jnp
from jax import lax
from jax.experimental import pallas as pl
from jax.experimental.pallas import tpu as pltpu
from jax.experimental.pallas import tpu_sc as plsc

_N = 10000
_E = 320000
_G = 64
_NC = 2
_NS = 16
_CH = 125
_RPT = 624
_RPT_LAST = _N - (_NS - 1) * _RPT
_GRP = 16

_BR = 2000
_NSTEPS = _N // _BR


def _make_sc_agg(split_edges: bool):
    if split_edges:
        nch = (_E // _NC) // _NS // _CH
        core_row_off = (_E // _NC) // _CH
    else:
        nch = _E // _NS // _CH
        core_row_off = 0
    ngrp = nch // _GRP

    def body(h_a, h_b, src_r, dst_r, zblk, out_a, out_b,
             acc, idxs, idxd, rows0, rows1, sem0, sem1, sem2, sem3, sem4):
        c = lax.axis_index("c")
        s = lax.axis_index("s")

        @pl.when(s < _NS - 1)
        def _():
            pltpu.sync_copy(zblk.at[pl.ds(0, _RPT)],
                            acc.at[pl.ds(s * _RPT, _RPT)])

        @pl.when(s == _NS - 1)
        def _():
            pltpu.sync_copy(zblk, acc.at[pl.ds((_NS - 1) * _RPT, _RPT_LAST)])

        row0 = c * core_row_off + s * nch
        plsc.subcore_barrier()

        def chunk(h):
            rbuf = (rows0, rows1)
            gsem = (sem0, sem1)
            ssem = (sem2, sem3)

            def gather_start(j, b):
                pltpu.async_copy(h.at[idxs.at[j]], rbuf[b], gsem[b])

            def gather_wait(j, b):
                pltpu.make_async_copy(h.at[idxs.at[j]], rbuf[b],
                                      gsem[b]).wait()

            def scatter_start(j, b):
                pltpu.async_copy(rbuf[b], acc.at[idxd.at[j]], ssem[b],
                                 add=True)

            def scatter_wait(j, b):
                pltpu.make_async_copy(rbuf[b], acc.at[idxd.at[j]],
                                      ssem[b]).wait()

            def group(g, carry):
                gr = row0 + g * _GRP
                pltpu.async_copy(src_r.at[pl.ds(gr, _GRP)], idxs, sem4)
                pltpu.async_copy(dst_r.at[pl.ds(gr, _GRP)], idxd, sem4)
                pltpu.make_async_copy(src_r.at[pl.ds(gr, _GRP)], idxs,
                                      sem4).wait()
                pltpu.make_async_copy(dst_r.at[pl.ds(gr, _GRP)], idxd,
                                      sem4).wait()
                gather_start(0, 0)
                for j in range(_GRP):
                    b = j % 2
                    if j + 1 < _GRP:
                        if j >= 1:
                            scatter_wait(j - 1, 1 - b)
                        gather_start(j + 1, 1 - b)
                    gather_wait(j, b)
                    scatter_start(j, b)
                    if j >= _GRP - 2:
                        scatter_wait(j, b)
                return carry
            lax.fori_loop(0, ngrp, group, 0)

        @pl.when(c == 0)
        def _():
            chunk(h_a)

        @pl.when(c == 1)
        def _():
            chunk(h_b)

        plsc.subcore_barrier()

        def copy_out(out):
            @pl.when(s < _NS - 1)
            def _():
                pltpu.sync_copy(acc.at[pl.ds(s * _RPT, _RPT)],
                                out.at[pl.ds(s * _RPT, _RPT)])

            @pl.when(s == _NS - 1)
            def _():
                pltpu.sync_copy(acc.at[pl.ds((_NS - 1) * _RPT, _RPT_LAST)],
                                out.at[pl.ds((_NS - 1) * _RPT, _RPT_LAST)])

        @pl.when(c == 0)
        def _():
            copy_out(out_a)

        @pl.when(c == 1)
        def _():
            copy_out(out_b)

    mesh = plsc.VectorSubcoreMesh(core_axis_name="c", subcore_axis_name="s")
    return pl.kernel(
        body,
        out_type=(jax.ShapeDtypeStruct((_N, 128), jnp.float32),
                  jax.ShapeDtypeStruct((_N, 128), jnp.float32)),
        mesh=mesh,
        scratch_types=[
            pltpu.VMEM_SHARED((_N, 128), jnp.float32),
            pltpu.VMEM((_GRP, _CH), jnp.int32),
            pltpu.VMEM((_GRP, _CH), jnp.int32),
            pltpu.VMEM((_CH, 128), jnp.float32),
            pltpu.VMEM((_CH, 128), jnp.float32),
            pltpu.SemaphoreType.DMA,
            pltpu.SemaphoreType.DMA,
            pltpu.SemaphoreType.DMA,
            pltpu.SemaphoreType.DMA,
            pltpu.SemaphoreType.DMA,
        ],
    )


def _pre0_body(x_ref, p0_ref, p1_ref, w1_ref, b1_ref, w2_ref, b2_ref,
               zpre_ref, stats_ref):
    i = pl.program_id(0)
    zin = x_ref[...] + p0_ref[...] + p1_ref[...]
    z1 = jnp.maximum(jnp.dot(zin, w1_ref[...]) + b1_ref[...], 0.0)
    zp = jnp.dot(z1, w2_ref[...]) + b2_ref[...]
    zpre_ref[...] = zp
    st = jnp.concatenate(
        [jnp.sum(zp, axis=0, keepdims=True),
         jnp.sum(zp * zp, axis=0, keepdims=True),
         jnp.zeros((6, zp.shape[1]), jnp.float32)], axis=0)

    @pl.when(i == 0)
    def _():
        stats_ref[...] = st

    @pl.when(i != 0)
    def _():
        stats_ref[...] = stats_ref[...] + st


def _pre12_body(h0_ref, h1_ref, a0_ref, a1_ref, w1_ref, b1_ref, w2_ref,
                b2_ref, zpre_ref, stats_ref):
    i = pl.program_id(0)
    zin0 = h0_ref[...] + a0_ref[...]
    zin1 = h1_ref[...] + a1_ref[...]
    z1 = jnp.maximum(
        jnp.dot(zin0, w1_ref[0:128, :]) + jnp.dot(zin1, w1_ref[128:256, :])
        + b1_ref[...], 0.0)
    zp = jnp.dot(z1, w2_ref[...]) + b2_ref[...]
    zpre_ref[...] = zp
    st = jnp.concatenate(
        [jnp.sum(zp, axis=0, keepdims=True),
         jnp.sum(zp * zp, axis=0, keepdims=True),
         jnp.zeros((6, zp.shape[1]), jnp.float32)], axis=0)

    @pl.when(i == 0)
    def _():
        stats_ref[...] = st

    @pl.when(i != 0)
    def _():
        stats_ref[...] = stats_ref[...] + st


def _row_block(din):
    return pl.BlockSpec((_BR, din), lambda i: (i, 0))


def _full_block(shape):
    return pl.BlockSpec(shape, lambda i: tuple(0 for _ in shape))


def _tc_pre0(x, p0, p1, w1, b1, w2, b2):
    return pl.pallas_call(
        _pre0_body,
        grid=(_NSTEPS,),
        in_specs=[_row_block(128), _row_block(128), _row_block(128),
                  _full_block((128, 256)), _full_block((1, 256)),
                  _full_block((256, 256)), _full_block((1, 256))],
        out_specs=[pl.BlockSpec((_BR, 256), lambda i: (i, 0)),
                   _full_block((8, 256))],
        out_shape=[jax.ShapeDtypeStruct((_N, 256), jnp.float32),
                   jax.ShapeDtypeStruct((8, 256), jnp.float32)],
    )(x, p0, p1, w1, b1, w2, b2)


def _tc_pre12(h0, h1, a0, a1, w1, b1, w2, b2):
    return pl.pallas_call(
        _pre12_body,
        grid=(_NSTEPS,),
        in_specs=[_row_block(128), _row_block(128), _row_block(128),
                  _row_block(128),
                  _full_block((256, 256)), _full_block((1, 256)),
                  _full_block((256, 256)), _full_block((1, 256))],
        out_specs=[pl.BlockSpec((_BR, 256), lambda i: (i, 0)),
                   _full_block((8, 256))],
        out_shape=[jax.ShapeDtypeStruct((_N, 256), jnp.float32),
                   jax.ShapeDtypeStruct((8, 256), jnp.float32)],
    )(h0, h1, a0, a1, w1, b1, w2, b2)


def _post_body_res(zpre_ref, stats_ref, gam_ref, bet_ref, h0_ref, h1_ref,
                   o0_ref, o1_ref):
    _post_common(zpre_ref, stats_ref, gam_ref, bet_ref, h0_ref, h1_ref,
                 o0_ref, o1_ref)


def _post_body_nores(zpre_ref, stats_ref, gam_ref, bet_ref, o0_ref, o1_ref):
    _post_common(zpre_ref, stats_ref, gam_ref, bet_ref, None, None,
                 o0_ref, o1_ref)


def _post_common(zpre_ref, stats_ref, gam_ref, bet_ref, h0_ref, h1_ref,
                 o0_ref, o1_ref):
    inv_n = 1.0 / _N
    mean = stats_ref[0:1, :] * inv_n
    ex2 = stats_ref[1:2, :] * inv_n
    var = ex2 - mean * mean
    scale = gam_ref[...] / jnp.sqrt(var + 1e-5)
    y = (zpre_ref[...] - mean) * scale + bet_ref[...]
    y = jnp.maximum(y, 0.0)
    y0 = y[:, 0:128]
    y1 = y[:, 128:256]
    if h0_ref is not None:
        y0 = y0 + h0_ref[...]
        y1 = y1 + h1_ref[...]
    o0_ref[...] = y0
    o1_ref[...] = y1


def _tc_post(zpre, stats, gam, bet, h0=None, h1=None):
    residual = h0 is not None
    in_specs = [pl.BlockSpec((_BR, 256), lambda i: (i, 0)),
                _full_block((8, 256)), _full_block((1, 256)),
                _full_block((1, 256))]
    args = [zpre, stats, gam, bet]
    if residual:
        in_specs += [_row_block(128), _row_block(128)]
        args += [h0, h1]
    return pl.pallas_call(
        _post_body_res if residual else _post_body_nores,
        grid=(_NSTEPS,),
        in_specs=in_specs,
        out_specs=[_row_block(128), _row_block(128)],
        out_shape=[jax.ShapeDtypeStruct((_N, 128), jnp.float32),
                   jax.ShapeDtypeStruct((_N, 128), jnp.float32)],
    )(*args)


def _pool_body(h0_ref, h1_ref, batch_ref, wm1_ref, bm1_ref, wm2_ref, bm2_ref,
               out_ref, sums, counts):
    i = pl.program_id(0)
    b = batch_ref[0]
    gids = lax.broadcasted_iota(jnp.int32, (_G, _BR), 0)
    mask = (gids == b).astype(jnp.float32)
    hcat = jnp.concatenate([h0_ref[...], h1_ref[...]], axis=1)
    part = jnp.dot(mask, hcat)
    cnt = jnp.broadcast_to(jnp.sum(mask, axis=1, keepdims=True), (_G, 128))

    @pl.when(i == 0)
    def _():
        sums[...] = part
        counts[...] = cnt

    @pl.when(i != 0)
    def _():
        sums[...] = sums[...] + part
        counts[...] = counts[...] + cnt

    @pl.when(i == _NSTEPS - 1)
    def _():
        hg = sums[...] / jnp.maximum(counts[:, 0:1], 1.0)
        t = jnp.maximum(jnp.dot(hg, wm1_ref[...]) + bm1_ref[...], 0.0)
        out_ref[...] = jnp.dot(t, wm2_ref[...]) + bm2_ref[...]


def _tc_pool_head(h0, h1, batch3, wm1, bm1, wm2p, bm2p):
    return pl.pallas_call(
        _pool_body,
        grid=(_NSTEPS,),
        in_specs=[_row_block(128), _row_block(128),
                  pl.BlockSpec((1, 1, _BR), lambda i: (i, 0, 0)),
                  _full_block((256, 256)), _full_block((1, 256)),
                  _full_block((256, 128)), _full_block((1, 128))],
        out_specs=_full_block((_G, 128)),
        out_shape=jax.ShapeDtypeStruct((_G, 128), jnp.float32),
        scratch_shapes=[pltpu.VMEM((_G, 256), jnp.float32),
                        pltpu.VMEM((_G, 128), jnp.float32)],
    )(h0, h1, batch3, wm1, bm1, wm2p, bm2p)


def kernel(x, edge_index, batch, W1_0, b1_0, W2_0, b2_0, gam_0, bet_0,
           W1_1, b1_1, W2_1, b2_1, gam_1, bet_1,
           W1_2, b1_2, W2_2, b2_2, gam_2, bet_2, Wm1, bm1, Wm2, bm2):
    src_r = edge_index[0].astype(jnp.int32).reshape(_E // _CH, _CH)
    dst_r = edge_index[1].astype(jnp.int32).reshape(_E // _CH, _CH)
    zblk = jnp.zeros((_RPT_LAST, 128), jnp.float32)

    r = lambda v: v.reshape(1, -1)
    agg_edges = _make_sc_agg(split_edges=True)
    agg_feats = _make_sc_agg(split_edges=False)

    p0, p1 = agg_edges(x, x, src_r, dst_r, zblk)
    zpre, stats = _tc_pre0(x, p0, p1, W1_0, r(b1_0), W2_0, r(b2_0))
    h0, h1 = _tc_post(zpre, stats, r(gam_0), r(bet_0))

    for (W1, b1, W2, b2, gam, bet) in (
            (W1_1, b1_1, W2_1, b2_1, gam_1, bet_1),
            (W1_2, b1_2, W2_2, b2_2, gam_2, bet_2)):
        a0, a1 = agg_feats(h0, h1, src_r, dst_r, zblk)
        zpre, stats = _tc_pre12(h0, h1, a0, a1, W1, r(b1), W2, r(b2))
        h0, h1 = _tc_post(zpre, stats, r(gam), r(bet), h0, h1)

    batch3 = batch.astype(jnp.int32).reshape(_NSTEPS, 1, _BR)
    wm2p = jnp.pad(Wm2, ((0, 0), (0, 118)))
    bm2p = jnp.pad(bm2, (0, 118)).reshape(1, 128)
    out = _tc_pool_head(h0, h1, batch3, Wm1, r(bm1), wm2p, bm2p)
    return out[:, :10]


# speedup vs baseline: 8.8311x; 1.0227x over previous
import functools

import jax
import jax.numpy as jnp
from jax import lax
from jax.experimental import pallas as pl
from jax.experimental.pallas import tpu as pltpu
from jax.experimental.pallas import tpu_sc as plsc

_N = 10000
_E = 320000
_G = 64
_NC = 2
_NS = 16
_CH = 125
_RPT = 624
_RPT_LAST = _N - (_NS - 1) * _RPT
_GRP = 16

_BR = 2000
_NSTEPS = _N // _BR


def _make_sc_agg(split_edges: bool):
    if split_edges:
        nch = (_E // _NC) // _NS // _CH
        core_row_off = (_E // _NC) // _CH
    else:
        nch = _E // _NS // _CH
        core_row_off = 0
    ngrp = nch // _GRP

    def body(h_a, h_b, src_r, dst_r, zblk, out_a, out_b,
             acc, idxs, idxd, rows0, rows1, sem0, sem1, sem2, sem3, sem4):
        c = lax.axis_index("c")
        s = lax.axis_index("s")

        @pl.when(s < _NS - 1)
        def _():
            pltpu.sync_copy(zblk.at[pl.ds(0, _RPT)],
                            acc.at[pl.ds(s * _RPT, _RPT)])

        @pl.when(s == _NS - 1)
        def _():
            pltpu.sync_copy(zblk, acc.at[pl.ds((_NS - 1) * _RPT, _RPT_LAST)])

        row0 = c * core_row_off + s * nch
        plsc.subcore_barrier()

        def chunk(h):
            rbuf = (rows0, rows1)
            gsem = (sem0, sem1)
            ssem = (sem2, sem3)

            def gather_start(j, b):
                pltpu.async_copy(h.at[idxs.at[j]], rbuf[b], gsem[b])

            def gather_wait(j, b):
                pltpu.make_async_copy(h.at[idxs.at[j]], rbuf[b],
                                      gsem[b]).wait()

            def scatter_start(j, b):
                pltpu.async_copy(rbuf[b], acc.at[idxd.at[j]], ssem[b],
                                 add=True)

            def scatter_wait(j, b):
                pltpu.make_async_copy(rbuf[b], acc.at[idxd.at[j]],
                                      ssem[b]).wait()

            def group(g, carry):
                gr = row0 + g * _GRP
                pltpu.async_copy(src_r.at[pl.ds(gr, _GRP)], idxs, sem4)
                pltpu.async_copy(dst_r.at[pl.ds(gr, _GRP)], idxd, sem4)
                pltpu.make_async_copy(src_r.at[pl.ds(gr, _GRP)], idxs,
                                      sem4).wait()
                pltpu.make_async_copy(dst_r.at[pl.ds(gr, _GRP)], idxd,
                                      sem4).wait()
                gather_start(0, 0)
                for j in range(_GRP):
                    b = j % 2
                    if j + 1 < _GRP:
                        if j >= 1:
                            scatter_wait(j - 1, 1 - b)
                        gather_start(j + 1, 1 - b)
                    gather_wait(j, b)
                    scatter_start(j, b)
                    if j >= _GRP - 2:
                        scatter_wait(j, b)
                return carry
            lax.fori_loop(0, ngrp, group, 0)

        @pl.when(c == 0)
        def _():
            chunk(h_a)

        @pl.when(c == 1)
        def _():
            chunk(h_b)

        plsc.subcore_barrier()

        def copy_out(out):
            @pl.when(s < _NS - 1)
            def _():
                pltpu.sync_copy(acc.at[pl.ds(s * _RPT, _RPT)],
                                out.at[pl.ds(s * _RPT, _RPT)])

            @pl.when(s == _NS - 1)
            def _():
                pltpu.sync_copy(acc.at[pl.ds((_NS - 1) * _RPT, _RPT_LAST)],
                                out.at[pl.ds((_NS - 1) * _RPT, _RPT_LAST)])

        @pl.when(c == 0)
        def _():
            copy_out(out_a)

        @pl.when(c == 1)
        def _():
            copy_out(out_b)

    mesh = plsc.VectorSubcoreMesh(core_axis_name="c", subcore_axis_name="s")
    return pl.kernel(
        body,
        out_type=(jax.ShapeDtypeStruct((_N, 128), jnp.float32),
                  jax.ShapeDtypeStruct((_N, 128), jnp.float32)),
        mesh=mesh,
        scratch_types=[
            pltpu.VMEM_SHARED((_N, 128), jnp.float32),
            pltpu.VMEM((_GRP, _CH), jnp.int32),
            pltpu.VMEM((_GRP, _CH), jnp.int32),
            pltpu.VMEM((_CH, 128), jnp.float32),
            pltpu.VMEM((_CH, 128), jnp.float32),
            pltpu.SemaphoreType.DMA,
            pltpu.SemaphoreType.DMA,
            pltpu.SemaphoreType.DMA,
            pltpu.SemaphoreType.DMA,
            pltpu.SemaphoreType.DMA,
        ],
    )


def _row_block(din):
    return pl.BlockSpec((_BR, din), lambda p, i: (i, 0))


def _agg_block():
    return pl.BlockSpec((_BR, 128), lambda p, i: (i * (1 - p), 0))


def _full_block(shape):
    return pl.BlockSpec(shape, lambda p, i: tuple(0 for _ in shape))


def _layer_common(phase, i, z1, z1scr, s1_ref, s2_ref):
    z1scr[pl.ds(i * _BR, _BR), :] = z1
    ss = jnp.concatenate(
        [jnp.sum(z1, axis=0, keepdims=True),
         jnp.zeros((7, z1.shape[1]), jnp.float32)], axis=0)
    s2c = lax.dot_general(z1, z1, (((0,), (0,)), ((), ())))

    @pl.when(i == 0)
    def _():
        s1_ref[...] = ss
        s2_ref[...] = s2c

    @pl.when(i != 0)
    def _():
        s1_ref[...] = s1_ref[...] + ss
        s2_ref[...] = s2_ref[...] + s2c


def _layer_finalize(w2_ref, b2_ref, gam_ref, bet_ref, s1_ref, s2_ref,
                    weff_ref, beff_ref):
    inv_n = 1.0 / _N
    w2 = w2_ref[...]
    a = jnp.dot(s2_ref[...] * inv_n, w2)
    t1 = jnp.sum(w2 * a, axis=0, keepdims=True)
    mu1 = s1_ref[0:1, :] * inv_n
    t2 = jnp.dot(mu1, w2)
    var = t1 - t2 * t2
    mean = t2 + b2_ref[...]
    scale = gam_ref[...] / jnp.sqrt(var + 1e-5)
    weff_ref[...] = w2 * scale
    beff_ref[...] = jnp.broadcast_to(
        (b2_ref[...] - mean) * scale + bet_ref[...], beff_ref.shape)


def _layer0_body(x_ref, p0_ref, p1_ref, w1_ref, b1_ref, w2_ref, b2_ref,
                 gam_ref, bet_ref, o0_ref, o1_ref,
                 z1scr, s1_ref, s2_ref, weff_ref, beff_ref):
    p = pl.program_id(0)
    i = pl.program_id(1)

    @pl.when(p == 0)
    def _():
        zin = x_ref[...] + p0_ref[...] + p1_ref[...]
        z1 = jnp.maximum(jnp.dot(zin, w1_ref[...]) + b1_ref[...], 0.0)
        _layer_common(p, i, z1, z1scr, s1_ref, s2_ref)

    @pl.when(p == 1)
    def _():
        @pl.when(i == 0)
        def _():
            _layer_finalize(w2_ref, b2_ref, gam_ref, bet_ref, s1_ref,
                            s2_ref, weff_ref, beff_ref)
        y = jnp.dot(z1scr[pl.ds(i * _BR, _BR), :], weff_ref[...]) \
            + beff_ref[0:1, :]
        y = jnp.maximum(y, 0.0)
        o0_ref[...] = y[:, 0:128]
        o1_ref[...] = y[:, 128:256]


def _layer12_body(h0_ref, h1_ref, a0_ref, a1_ref, w1_ref, b1_ref, w2_ref,
                  b2_ref, gam_ref, bet_ref, o0_ref, o1_ref,
                  z1scr, s1_ref, s2_ref, weff_ref, beff_ref):
    p = pl.program_id(0)
    i = pl.program_id(1)

    @pl.when(p == 0)
    def _():
        zin0 = h0_ref[...] + a0_ref[...]
        zin1 = h1_ref[...] + a1_ref[...]
        z1 = jnp.maximum(
            jnp.dot(zin0, w1_ref[0:128, :])
            + jnp.dot(zin1, w1_ref[128:256, :]) + b1_ref[...], 0.0)
        _layer_common(p, i, z1, z1scr, s1_ref, s2_ref)

    @pl.when(p == 1)
    def _():
        @pl.when(i == 0)
        def _():
            _layer_finalize(w2_ref, b2_ref, gam_ref, bet_ref, s1_ref,
                            s2_ref, weff_ref, beff_ref)
        y = jnp.dot(z1scr[pl.ds(i * _BR, _BR), :], weff_ref[...]) \
            + beff_ref[0:1, :]
        y = jnp.maximum(y, 0.0)
        o0_ref[...] = y[:, 0:128] + h0_ref[...]
        o1_ref[...] = y[:, 128:256] + h1_ref[...]


_LAYER_SCRATCH = [
    pltpu.VMEM((_N, 256), jnp.float32),
    pltpu.VMEM((8, 256), jnp.float32),
    pltpu.VMEM((256, 256), jnp.float32),
    pltpu.VMEM((256, 256), jnp.float32),
    pltpu.VMEM((8, 256), jnp.float32),
]

_LAYER_OUT = dict(
    out_specs=[pl.BlockSpec((_BR, 128), lambda p, i: (i, 0)),
               pl.BlockSpec((_BR, 128), lambda p, i: (i, 0))],
    out_shape=[jax.ShapeDtypeStruct((_N, 128), jnp.float32),
               jax.ShapeDtypeStruct((_N, 128), jnp.float32)],
    scratch_shapes=_LAYER_SCRATCH,
)


def _tc_layer0(x, p0, p1, w1, b1, w2, b2, gam, bet):
    return pl.pallas_call(
        _layer0_body,
        grid=(2, _NSTEPS),
        in_specs=[_row_block(128), _agg_block(), _agg_block(),
                  _full_block((128, 256)), _full_block((1, 256)),
                  _full_block((256, 256)), _full_block((1, 256)),
                  _full_block((1, 256)), _full_block((1, 256))],
        **_LAYER_OUT,
    )(x, p0, p1, w1, b1, w2, b2, gam, bet)


def _tc_layer12(h0, h1, a0, a1, w1, b1, w2, b2, gam, bet):
    return pl.pallas_call(
        _layer12_body,
        grid=(2, _NSTEPS),
        in_specs=[_row_block(128), _row_block(128), _agg_block(),
                  _agg_block(),
                  _full_block((256, 256)), _full_block((1, 256)),
                  _full_block((256, 256)), _full_block((1, 256)),
                  _full_block((1, 256)), _full_block((1, 256))],
        **_LAYER_OUT,
    )(h0, h1, a0, a1, w1, b1, w2, b2, gam, bet)


def _pool_body(h0_ref, h1_ref, batch_ref, wm1_ref, bm1_ref, wm2_ref, bm2_ref,
               out_ref, sums, counts):
    i = pl.program_id(0)
    b = batch_ref[0]
    gids = lax.broadcasted_iota(jnp.int32, (_G, _BR), 0)
    mask = (gids == b).astype(jnp.float32)
    hcat = jnp.concatenate([h0_ref[...], h1_ref[...]], axis=1)
    part = jnp.dot(mask, hcat)
    cnt = jnp.broadcast_to(jnp.sum(mask, axis=1, keepdims=True), (_G, 128))

    @pl.when(i == 0)
    def _():
        sums[...] = part
        counts[...] = cnt

    @pl.when(i != 0)
    def _():
        sums[...] = sums[...] + part
        counts[...] = counts[...] + cnt

    @pl.when(i == _NSTEPS - 1)
    def _():
        hg = sums[...] / jnp.maximum(counts[:, 0:1], 1.0)
        t = jnp.maximum(jnp.dot(hg, wm1_ref[...]) + bm1_ref[...], 0.0)
        out_ref[...] = jnp.dot(t, wm2_ref[...]) + bm2_ref[...]


def _tc_pool_head(h0, h1, batch3, wm1, bm1, wm2p, bm2p):
    rb = pl.BlockSpec((_BR, 128), lambda i: (i, 0))
    fb = lambda shape: pl.BlockSpec(shape, lambda i: tuple(0 for _ in shape))
    return pl.pallas_call(
        _pool_body,
        grid=(_NSTEPS,),
        in_specs=[rb, rb,
                  pl.BlockSpec((1, 1, _BR), lambda i: (i, 0, 0)),
                  fb((256, 256)), fb((1, 256)),
                  fb((256, 128)), fb((1, 128))],
        out_specs=fb((_G, 128)),
        out_shape=jax.ShapeDtypeStruct((_G, 128), jnp.float32),
        scratch_shapes=[pltpu.VMEM((_G, 256), jnp.float32),
                        pltpu.VMEM((_G, 128), jnp.float32)],
    )(h0, h1, batch3, wm1, bm1, wm2p, bm2p)


def kernel(x, edge_index, batch, W1_0, b1_0, W2_0, b2_0, gam_0, bet_0,
           W1_1, b1_1, W2_1, b2_1, gam_1, bet_1,
           W1_2, b1_2, W2_2, b2_2, gam_2, bet_2, Wm1, bm1, Wm2, bm2):
    src_r = edge_index[0].astype(jnp.int32).reshape(_E // _CH, _CH)
    dst_r = edge_index[1].astype(jnp.int32).reshape(_E // _CH, _CH)
    zblk = jnp.zeros((_RPT_LAST, 128), jnp.float32)

    r = lambda v: v.reshape(1, -1)
    agg_edges = _make_sc_agg(split_edges=True)
    agg_feats = _make_sc_agg(split_edges=False)

    p0, p1 = agg_edges(x, x, src_r, dst_r, zblk)
    h0, h1 = _tc_layer0(x, p0, p1, W1_0, r(b1_0), W2_0, r(b2_0),
                        r(gam_0), r(bet_0))

    for (W1, b1, W2, b2, gam, bet) in (
            (W1_1, b1_1, W2_1, b2_1, gam_1, bet_1),
            (W1_2, b1_2, W2_2, b2_2, gam_2, bet_2)):
        a0, a1 = agg_feats(h0, h1, src_r, dst_r, zblk)
        h0, h1 = _tc_layer12(h0, h1, a0, a1, W1, r(b1), W2, r(b2),
                             r(gam), r(bet))

    batch3 = batch.astype(jnp.int32).reshape(_NSTEPS, 1, _BR)
    wm2p = jnp.pad(Wm2, ((0, 0), (0, 118)))
    bm2p = jnp.pad(bm2, (0, 118)).reshape(1, 128)
    out = _tc_pool_head(h0, h1, batch3, Wm1, r(bm1), wm2p, bm2p)
    return out[:, :10]


# speedup vs baseline: 8.8316x; 1.0001x over previous
import functools

import jax
import jax.numpy as jnp
from jax import lax
from jax.experimental import pallas as pl
from jax.experimental.pallas import tpu as pltpu
from jax.experimental.pallas import tpu_sc as plsc

_N = 10000
_E = 320000
_G = 64
_NC = 2
_NS = 16
_CH = 125
_RPT = 624
_RPT_LAST = _N - (_NS - 1) * _RPT
_GRP = 16

_BR = 2000
_NSTEPS = _N // _BR


def _make_sc_agg(split_edges: bool):
    if split_edges:
        nch = (_E // _NC) // _NS // _CH
        core_row_off = (_E // _NC) // _CH
    else:
        nch = _E // _NS // _CH
        core_row_off = 0
    ngrp = nch // _GRP

    def body(h_a, h_b, src_r, dst_r, zblk, out_a, out_b,
             acc, idxs, idxd, rows0, rows1, sem0, sem1, sem2, sem3, sem4):
        c = lax.axis_index("c")
        s = lax.axis_index("s")

        @pl.when(s < _NS - 1)
        def _():
            pltpu.sync_copy(zblk.at[pl.ds(0, _RPT)],
                            acc.at[pl.ds(s * _RPT, _RPT)])

        @pl.when(s == _NS - 1)
        def _():
            pltpu.sync_copy(zblk, acc.at[pl.ds((_NS - 1) * _RPT, _RPT_LAST)])

        row0 = c * core_row_off + s * nch
        plsc.subcore_barrier()

        def chunk(h):
            rbuf = (rows0, rows1)
            gsem = (sem0, sem1)
            ssem = (sem2, sem3)

            def gather_start(j, b):
                pltpu.async_copy(h.at[idxs.at[j]], rbuf[b], gsem[b])

            def gather_wait(j, b):
                pltpu.make_async_copy(h.at[idxs.at[j]], rbuf[b],
                                      gsem[b]).wait()

            def scatter_start(j, b):
                pltpu.async_copy(rbuf[b], acc.at[idxd.at[j]], ssem[b],
                                 add=True)

            def scatter_wait(j, b):
                pltpu.make_async_copy(rbuf[b], acc.at[idxd.at[j]],
                                      ssem[b]).wait()

            def group(g, carry):
                gr = row0 + g * _GRP
                pltpu.async_copy(src_r.at[pl.ds(gr, _GRP)], idxs, sem4)
                pltpu.async_copy(dst_r.at[pl.ds(gr, _GRP)], idxd, sem4)
                pltpu.make_async_copy(src_r.at[pl.ds(gr, _GRP)], idxs,
                                      sem4).wait()
                pltpu.make_async_copy(dst_r.at[pl.ds(gr, _GRP)], idxd,
                                      sem4).wait()
                gather_start(0, 0)
                for j in range(_GRP):
                    b = j % 2
                    if j + 1 < _GRP:
                        if j >= 1:
                            scatter_wait(j - 1, 1 - b)
                        gather_start(j + 1, 1 - b)
                    gather_wait(j, b)
                    scatter_start(j, b)
                    if j >= _GRP - 2:
                        scatter_wait(j, b)
                return carry
            lax.fori_loop(0, ngrp, group, 0)

        @pl.when(c == 0)
        def _():
            chunk(h_a)

        @pl.when(c == 1)
        def _():
            chunk(h_b)

        plsc.subcore_barrier()

        def copy_out(out):
            @pl.when(s < _NS - 1)
            def _():
                pltpu.sync_copy(acc.at[pl.ds(s * _RPT, _RPT)],
                                out.at[pl.ds(s * _RPT, _RPT)])

            @pl.when(s == _NS - 1)
            def _():
                pltpu.sync_copy(acc.at[pl.ds((_NS - 1) * _RPT, _RPT_LAST)],
                                out.at[pl.ds((_NS - 1) * _RPT, _RPT_LAST)])

        @pl.when(c == 0)
        def _():
            copy_out(out_a)

        @pl.when(c == 1)
        def _():
            copy_out(out_b)

    mesh = plsc.VectorSubcoreMesh(core_axis_name="c", subcore_axis_name="s")
    return pl.kernel(
        body,
        out_type=(jax.ShapeDtypeStruct((_N, 128), jnp.float32),
                  jax.ShapeDtypeStruct((_N, 128), jnp.float32)),
        mesh=mesh,
        scratch_types=[
            pltpu.VMEM_SHARED((_N, 128), jnp.float32),
            pltpu.VMEM((_GRP, _CH), jnp.int32),
            pltpu.VMEM((_GRP, _CH), jnp.int32),
            pltpu.VMEM((_CH, 128), jnp.float32),
            pltpu.VMEM((_CH, 128), jnp.float32),
            pltpu.SemaphoreType.DMA,
            pltpu.SemaphoreType.DMA,
            pltpu.SemaphoreType.DMA,
            pltpu.SemaphoreType.DMA,
            pltpu.SemaphoreType.DMA,
        ],
    )


def _row_block(din):
    return pl.BlockSpec((_BR, din), lambda p, i: (i, 0))


def _agg_block():
    return pl.BlockSpec((_BR, 128), lambda p, i: (i * (1 - p), 0))


def _full_block(shape):
    return pl.BlockSpec(shape, lambda p, i: tuple(0 for _ in shape))


def _layer_common(i, zp, zscr, s1_ref):
    zscr[pl.ds(i * _BR, _BR), :] = zp
    ss = jnp.concatenate(
        [jnp.sum(zp, axis=0, keepdims=True),
         jnp.sum(zp * zp, axis=0, keepdims=True),
         jnp.zeros((6, zp.shape[1]), jnp.float32)], axis=0)

    @pl.when(i == 0)
    def _():
        s1_ref[...] = ss

    @pl.when(i != 0)
    def _():
        s1_ref[...] = s1_ref[...] + ss


def _layer_finalize(gam_ref, bet_ref, s1_ref, coef_ref):
    inv_n = 1.0 / _N
    mean = s1_ref[0:1, :] * inv_n
    var = s1_ref[1:2, :] * inv_n - mean * mean
    scale = gam_ref[...] / jnp.sqrt(var + 1e-5)
    shift = bet_ref[...] - mean * scale
    coef_ref[...] = jnp.concatenate(
        [scale, shift, jnp.zeros((6, scale.shape[1]), jnp.float32)], axis=0)


def _layer0_body(x_ref, p0_ref, p1_ref, w1_ref, b1_ref, w2_ref, b2_ref,
                 gam_ref, bet_ref, o0_ref, o1_ref,
                 zscr, s1_ref, coef_ref):
    p = pl.program_id(0)
    i = pl.program_id(1)

    @pl.when(p == 0)
    def _():
        zin = x_ref[...] + p0_ref[...] + p1_ref[...]
        z1 = jnp.maximum(jnp.dot(zin, w1_ref[...]) + b1_ref[...], 0.0)
        zp = jnp.dot(z1, w2_ref[...]) + b2_ref[...]
        _layer_common(i, zp, zscr, s1_ref)

    @pl.when(p == 1)
    def _():
        @pl.when(i == 0)
        def _():
            _layer_finalize(gam_ref, bet_ref, s1_ref, coef_ref)
        y = zscr[pl.ds(i * _BR, _BR), :] * coef_ref[0:1, :] \
            + coef_ref[1:2, :]
        y = jnp.maximum(y, 0.0)
        o0_ref[...] = y[:, 0:128]
        o1_ref[...] = y[:, 128:256]


def _layer12_body(h0_ref, h1_ref, a0_ref, a1_ref, w1_ref, b1_ref, w2_ref,
                  b2_ref, gam_ref, bet_ref, o0_ref, o1_ref,
                  zscr, s1_ref, coef_ref):
    p = pl.program_id(0)
    i = pl.program_id(1)

    @pl.when(p == 0)
    def _():
        zin0 = h0_ref[...] + a0_ref[...]
        zin1 = h1_ref[...] + a1_ref[...]
        z1 = jnp.maximum(
            jnp.dot(zin0, w1_ref[0:128, :])
            + jnp.dot(zin1, w1_ref[128:256, :]) + b1_ref[...], 0.0)
        zp = jnp.dot(z1, w2_ref[...]) + b2_ref[...]
        _layer_common(i, zp, zscr, s1_ref)

    @pl.when(p == 1)
    def _():
        @pl.when(i == 0)
        def _():
            _layer_finalize(gam_ref, bet_ref, s1_ref, coef_ref)
        y = zscr[pl.ds(i * _BR, _BR), :] * coef_ref[0:1, :] \
            + coef_ref[1:2, :]
        y = jnp.maximum(y, 0.0)
        o0_ref[...] = y[:, 0:128] + h0_ref[...]
        o1_ref[...] = y[:, 128:256] + h1_ref[...]


_LAYER_SCRATCH = [
    pltpu.VMEM((_N, 256), jnp.float32),
    pltpu.VMEM((8, 256), jnp.float32),
    pltpu.VMEM((8, 256), jnp.float32),
]

_LAYER_OUT = dict(
    out_specs=[pl.BlockSpec((_BR, 128), lambda p, i: (i, 0)),
               pl.BlockSpec((_BR, 128), lambda p, i: (i, 0))],
    out_shape=[jax.ShapeDtypeStruct((_N, 128), jnp.float32),
               jax.ShapeDtypeStruct((_N, 128), jnp.float32)],
    scratch_shapes=_LAYER_SCRATCH,
)


def _tc_layer0(x, p0, p1, w1, b1, w2, b2, gam, bet):
    return pl.pallas_call(
        _layer0_body,
        grid=(2, _NSTEPS),
        in_specs=[_row_block(128), _agg_block(), _agg_block(),
                  _full_block((128, 256)), _full_block((1, 256)),
                  _full_block((256, 256)), _full_block((1, 256)),
                  _full_block((1, 256)), _full_block((1, 256))],
        **_LAYER_OUT,
    )(x, p0, p1, w1, b1, w2, b2, gam, bet)


def _tc_layer12(h0, h1, a0, a1, w1, b1, w2, b2, gam, bet):
    return pl.pallas_call(
        _layer12_body,
        grid=(2, _NSTEPS),
        in_specs=[_row_block(128), _row_block(128), _agg_block(),
                  _agg_block(),
                  _full_block((256, 256)), _full_block((1, 256)),
                  _full_block((256, 256)), _full_block((1, 256)),
                  _full_block((1, 256)), _full_block((1, 256))],
        **_LAYER_OUT,
    )(h0, h1, a0, a1, w1, b1, w2, b2, gam, bet)


def _pool_body(h0_ref, h1_ref, batch_ref, wm1_ref, bm1_ref, wm2_ref, bm2_ref,
               out_ref, sums, counts):
    i = pl.program_id(0)
    b = batch_ref[0]
    gids = lax.broadcasted_iota(jnp.int32, (_G, _BR), 0)
    mask = (gids == b).astype(jnp.float32)
    hcat = jnp.concatenate([h0_ref[...], h1_ref[...]], axis=1)
    part = jnp.dot(mask, hcat)
    cnt = jnp.broadcast_to(jnp.sum(mask, axis=1, keepdims=True), (_G, 128))

    @pl.when(i == 0)
    def _():
        sums[...] = part
        counts[...] = cnt

    @pl.when(i != 0)
    def _():
        sums[...] = sums[...] + part
        counts[...] = counts[...] + cnt

    @pl.when(i == _NSTEPS - 1)
    def _():
        hg = sums[...] / jnp.maximum(counts[:, 0:1], 1.0)
        t = jnp.maximum(jnp.dot(hg, wm1_ref[...]) + bm1_ref[...], 0.0)
        out_ref[...] = jnp.dot(t, wm2_ref[...]) + bm2_ref[...]


def _tc_pool_head(h0, h1, batch3, wm1, bm1, wm2p, bm2p):
    rb = pl.BlockSpec((_BR, 128), lambda i: (i, 0))
    fb = lambda shape: pl.BlockSpec(shape, lambda i: tuple(0 for _ in shape))
    return pl.pallas_call(
        _pool_body,
        grid=(_NSTEPS,),
        in_specs=[rb, rb,
                  pl.BlockSpec((1, 1, _BR), lambda i: (i, 0, 0)),
                  fb((256, 256)), fb((1, 256)),
                  fb((256, 128)), fb((1, 128))],
        out_specs=fb((_G, 128)),
        out_shape=jax.ShapeDtypeStruct((_G, 128), jnp.float32),
        scratch_shapes=[pltpu.VMEM((_G, 256), jnp.float32),
                        pltpu.VMEM((_G, 128), jnp.float32)],
    )(h0, h1, batch3, wm1, bm1, wm2p, bm2p)


def kernel(x, edge_index, batch, W1_0, b1_0, W2_0, b2_0, gam_0, bet_0,
           W1_1, b1_1, W2_1, b2_1, gam_1, bet_1,
           W1_2, b1_2, W2_2, b2_2, gam_2, bet_2, Wm1, bm1, Wm2, bm2):
    src_r = edge_index[0].astype(jnp.int32).reshape(_E // _CH, _CH)
    dst_r = edge_index[1].astype(jnp.int32).reshape(_E // _CH, _CH)
    zblk = jnp.zeros((_RPT_LAST, 128), jnp.float32)

    r = lambda v: v.reshape(1, -1)
    agg_edges = _make_sc_agg(split_edges=True)
    agg_feats = _make_sc_agg(split_edges=False)

    p0, p1 = agg_edges(x, x, src_r, dst_r, zblk)
    h0, h1 = _tc_layer0(x, p0, p1, W1_0, r(b1_0), W2_0, r(b2_0),
                        r(gam_0), r(bet_0))

    for (W1, b1, W2, b2, gam, bet) in (
            (W1_1, b1_1, W2_1, b2_1, gam_1, bet_1),
            (W1_2, b1_2, W2_2, b2_2, gam_2, bet_2)):
        a0, a1 = agg_feats(h0, h1, src_r, dst_r, zblk)
        h0, h1 = _tc_layer12(h0, h1, a0, a1, W1, r(b1), W2, r(b2),
                             r(gam), r(bet))

    batch3 = batch.astype(jnp.int32).reshape(_NSTEPS, 1, _BR)
    wm2p = jnp.pad(Wm2, ((0, 0), (0, 118)))
    bm2p = jnp.pad(bm2, (0, 118)).reshape(1, 128)
    out = _tc_pool_head(h0, h1, batch3, Wm1, r(bm1), wm2p, bm2p)
    return out[:, :10]


# speedup vs baseline: 8.9901x; 1.0180x over previous
import functools

import jax
import jax.numpy as jnp
from jax import lax
from jax.experimental import pallas as pl
from jax.experimental.pallas import tpu as pltpu
from jax.experimental.pallas import tpu_sc as plsc

_N = 10000
_E = 320000
_G = 64
_NC = 2
_NS = 16
_CH = 125
_RPT = 624
_RPT_LAST = _N - (_NS - 1) * _RPT
_GRP = 16

_BR = 2000
_NSTEPS = _N // _BR


def _make_sc_agg(split_edges: bool):
    if split_edges:
        nch = (_E // _NC) // _NS // _CH
        core_row_off = (_E // _NC) // _CH
    else:
        nch = _E // _NS // _CH
        core_row_off = 0
    ngrp = nch // _GRP

    def body(h_a, h_b, src_r, dst_r, zblk, out_a, out_b,
             acc, idxs, idxd, rows0, rows1, sem0, sem1, sem2, sem3, sem4):
        c = lax.axis_index("c")
        s = lax.axis_index("s")

        @pl.when(s < _NS - 1)
        def _():
            pltpu.sync_copy(zblk.at[pl.ds(0, _RPT)],
                            acc.at[pl.ds(s * _RPT, _RPT)])

        @pl.when(s == _NS - 1)
        def _():
            pltpu.sync_copy(zblk, acc.at[pl.ds((_NS - 1) * _RPT, _RPT_LAST)])

        row0 = c * core_row_off + s * nch
        plsc.subcore_barrier()

        def chunk(h):
            rbuf = (rows0, rows1)
            gsem = (sem0, sem1)
            ssem = (sem2, sem3)

            def gather_start(j, b):
                pltpu.async_copy(h.at[idxs.at[j]], rbuf[b], gsem[b])

            def gather_wait(j, b):
                pltpu.make_async_copy(h.at[idxs.at[j]], rbuf[b],
                                      gsem[b]).wait()

            def scatter_start(j, b):
                pltpu.async_copy(rbuf[b], acc.at[idxd.at[j]], ssem[b],
                                 add=True)

            def scatter_wait(j, b):
                pltpu.make_async_copy(rbuf[b], acc.at[idxd.at[j]],
                                      ssem[b]).wait()

            def group(g, carry):
                gr = row0 + g * _GRP
                pltpu.async_copy(src_r.at[pl.ds(gr, _GRP)], idxs, sem4)
                pltpu.async_copy(dst_r.at[pl.ds(gr, _GRP)], idxd, sem4)
                pltpu.make_async_copy(src_r.at[pl.ds(gr, _GRP)], idxs,
                                      sem4).wait()
                pltpu.make_async_copy(dst_r.at[pl.ds(gr, _GRP)], idxd,
                                      sem4).wait()
                gather_start(0, 0)
                for j in range(_GRP):
                    b = j % 2
                    if j + 1 < _GRP:
                        if j >= 1:
                            scatter_wait(j - 1, 1 - b)
                        gather_start(j + 1, 1 - b)
                    gather_wait(j, b)
                    scatter_start(j, b)
                    if j >= _GRP - 2:
                        scatter_wait(j, b)
                return carry
            lax.fori_loop(0, ngrp, group, 0)

        @pl.when(c == 0)
        def _():
            chunk(h_a)

        @pl.when(c == 1)
        def _():
            chunk(h_b)

        plsc.subcore_barrier()

        def copy_out(out):
            @pl.when(s < _NS - 1)
            def _():
                pltpu.sync_copy(acc.at[pl.ds(s * _RPT, _RPT)],
                                out.at[pl.ds(s * _RPT, _RPT)])

            @pl.when(s == _NS - 1)
            def _():
                pltpu.sync_copy(acc.at[pl.ds((_NS - 1) * _RPT, _RPT_LAST)],
                                out.at[pl.ds((_NS - 1) * _RPT, _RPT_LAST)])

        @pl.when(c == 0)
        def _():
            copy_out(out_a)

        @pl.when(c == 1)
        def _():
            copy_out(out_b)

    mesh = plsc.VectorSubcoreMesh(core_axis_name="c", subcore_axis_name="s")
    return pl.kernel(
        body,
        out_type=(jax.ShapeDtypeStruct((_N, 128), jnp.float32),
                  jax.ShapeDtypeStruct((_N, 128), jnp.float32)),
        mesh=mesh,
        scratch_types=[
            pltpu.VMEM_SHARED((_N, 128), jnp.float32),
            pltpu.VMEM((_GRP, _CH), jnp.int32),
            pltpu.VMEM((_GRP, _CH), jnp.int32),
            pltpu.VMEM((_CH, 128), jnp.float32),
            pltpu.VMEM((_CH, 128), jnp.float32),
            pltpu.SemaphoreType.DMA,
            pltpu.SemaphoreType.DMA,
            pltpu.SemaphoreType.DMA,
            pltpu.SemaphoreType.DMA,
            pltpu.SemaphoreType.DMA,
        ],
    )


def _row_block(din):
    return pl.BlockSpec((_BR, din), lambda p, i: (i, 0))


def _agg_block():
    return pl.BlockSpec((_BR, 128), lambda p, i: (i * (1 - p), 0))


def _full_block(shape):
    return pl.BlockSpec(shape, lambda p, i: tuple(0 for _ in shape))


def _layer_common(i, zp, zscr, s1_ref):
    zscr[pl.ds(i * _BR, _BR), :] = zp
    ss = jnp.concatenate(
        [jnp.sum(zp, axis=0, keepdims=True),
         jnp.sum(zp * zp, axis=0, keepdims=True),
         jnp.zeros((6, zp.shape[1]), jnp.float32)], axis=0)

    @pl.when(i == 0)
    def _():
        s1_ref[...] = ss

    @pl.when(i != 0)
    def _():
        s1_ref[...] = s1_ref[...] + ss


def _layer_finalize(gam_ref, bet_ref, s1_ref, coef_ref):
    inv_n = 1.0 / _N
    mean = s1_ref[0:1, :] * inv_n
    var = s1_ref[1:2, :] * inv_n - mean * mean
    scale = gam_ref[...] / jnp.sqrt(var + 1e-5)
    shift = bet_ref[...] - mean * scale
    coef_ref[...] = jnp.concatenate(
        [scale, shift, jnp.zeros((6, scale.shape[1]), jnp.float32)], axis=0)


def _layer0_body(x_ref, p0_ref, p1_ref, w1_ref, b1_ref, w2_ref, b2_ref,
                 gam_ref, bet_ref, o0_ref, o1_ref,
                 zscr, s1_ref, coef_ref):
    p = pl.program_id(0)
    i = pl.program_id(1)

    @pl.when(p == 0)
    def _():
        zin = x_ref[...] + p0_ref[...] + p1_ref[...]
        z1 = jnp.maximum(jnp.dot(zin, w1_ref[...]) + b1_ref[...], 0.0)
        zp = jnp.dot(z1, w2_ref[...]) + b2_ref[...]
        _layer_common(i, zp, zscr, s1_ref)

    @pl.when(p == 1)
    def _():
        @pl.when(i == 0)
        def _():
            _layer_finalize(gam_ref, bet_ref, s1_ref, coef_ref)
        y = zscr[pl.ds(i * _BR, _BR), :] * coef_ref[0:1, :] \
            + coef_ref[1:2, :]
        y = jnp.maximum(y, 0.0)
        o0_ref[...] = y[:, 0:128]
        o1_ref[...] = y[:, 128:256]


def _layer12_body(h0_ref, h1_ref, a0_ref, a1_ref, w1_ref, b1_ref, w2_ref,
                  b2_ref, gam_ref, bet_ref, o0_ref, o1_ref,
                  zscr, s1_ref, coef_ref):
    p = pl.program_id(0)
    i = pl.program_id(1)

    @pl.when(p == 0)
    def _():
        zin0 = h0_ref[...] + a0_ref[...]
        zin1 = h1_ref[...] + a1_ref[...]
        z1 = jnp.maximum(
            jnp.dot(zin0, w1_ref[0:128, :])
            + jnp.dot(zin1, w1_ref[128:256, :]) + b1_ref[...], 0.0)
        zp = jnp.dot(z1, w2_ref[...]) + b2_ref[...]
        _layer_common(i, zp, zscr, s1_ref)

    @pl.when(p == 1)
    def _():
        @pl.when(i == 0)
        def _():
            _layer_finalize(gam_ref, bet_ref, s1_ref, coef_ref)
        y = zscr[pl.ds(i * _BR, _BR), :] * coef_ref[0:1, :] \
            + coef_ref[1:2, :]
        y = jnp.maximum(y, 0.0)
        o0_ref[...] = y[:, 0:128] + h0_ref[...]
        o1_ref[...] = y[:, 128:256] + h1_ref[...]


def _layer2_pool_body(h0_ref, h1_ref, a0_ref, a1_ref, w1_ref, b1_ref,
                      w2_ref, b2_ref, gam_ref, bet_ref, batch_ref,
                      wm1_ref, bm1_ref, wm2_ref, bm2_ref, out_ref,
                      zscr, s1_ref, coef_ref, sums, counts):
    p = pl.program_id(0)
    i = pl.program_id(1)

    @pl.when(p == 0)
    def _():
        zin0 = h0_ref[...] + a0_ref[...]
        zin1 = h1_ref[...] + a1_ref[...]
        z1 = jnp.maximum(
            jnp.dot(zin0, w1_ref[0:128, :])
            + jnp.dot(zin1, w1_ref[128:256, :]) + b1_ref[...], 0.0)
        zp = jnp.dot(z1, w2_ref[...]) + b2_ref[...]
        _layer_common(i, zp, zscr, s1_ref)

    @pl.when(p == 1)
    def _():
        @pl.when(i == 0)
        def _():
            _layer_finalize(gam_ref, bet_ref, s1_ref, coef_ref)
        y = zscr[pl.ds(i * _BR, _BR), :] * coef_ref[0:1, :] \
            + coef_ref[1:2, :]
        y = jnp.maximum(y, 0.0)
        y0 = y[:, 0:128] + h0_ref[...]
        y1 = y[:, 128:256] + h1_ref[...]
        b = batch_ref[0]
        gids = lax.broadcasted_iota(jnp.int32, (_G, _BR), 0)
        mask = (gids == b).astype(jnp.float32)
        hcat = jnp.concatenate([y0, y1], axis=1)
        part = jnp.dot(mask, hcat)
        cnt = jnp.broadcast_to(jnp.sum(mask, axis=1, keepdims=True),
                               (_G, 128))

        @pl.when(i == 0)
        def _():
            sums[...] = part
            counts[...] = cnt

        @pl.when(i != 0)
        def _():
            sums[...] = sums[...] + part
            counts[...] = counts[...] + cnt

        @pl.when(i == _NSTEPS - 1)
        def _():
            hg = sums[...] / jnp.maximum(counts[:, 0:1], 1.0)
            t = jnp.maximum(jnp.dot(hg, wm1_ref[...]) + bm1_ref[...], 0.0)
            out_ref[...] = jnp.dot(t, wm2_ref[...]) + bm2_ref[...]


_LAYER_SCRATCH = [
    pltpu.VMEM((_N, 256), jnp.float32),
    pltpu.VMEM((8, 256), jnp.float32),
    pltpu.VMEM((8, 256), jnp.float32),
]


def _tc_layer2_pool(h0, h1, a0, a1, w1, b1, w2, b2, gam, bet, batch3,
                    wm1, bm1, wm2p, bm2p):
    return pl.pallas_call(
        _layer2_pool_body,
        grid=(2, _NSTEPS),
        in_specs=[_row_block(128), _row_block(128), _agg_block(),
                  _agg_block(),
                  _full_block((256, 256)), _full_block((1, 256)),
                  _full_block((256, 256)), _full_block((1, 256)),
                  _full_block((1, 256)), _full_block((1, 256)),
                  pl.BlockSpec((1, 1, _BR), lambda p, i: (i * p, 0, 0)),
                  _full_block((256, 256)), _full_block((1, 256)),
                  _full_block((256, 128)), _full_block((1, 128))],
        out_specs=_full_block((_G, 128)),
        out_shape=jax.ShapeDtypeStruct((_G, 128), jnp.float32),
        scratch_shapes=_LAYER_SCRATCH + [
            pltpu.VMEM((_G, 256), jnp.float32),
            pltpu.VMEM((_G, 128), jnp.float32)],
    )(h0, h1, a0, a1, w1, b1, w2, b2, gam, bet, batch3,
      wm1, bm1, wm2p, bm2p)

_LAYER_OUT = dict(
    out_specs=[pl.BlockSpec((_BR, 128), lambda p, i: (i, 0)),
               pl.BlockSpec((_BR, 128), lambda p, i: (i, 0))],
    out_shape=[jax.ShapeDtypeStruct((_N, 128), jnp.float32),
               jax.ShapeDtypeStruct((_N, 128), jnp.float32)],
    scratch_shapes=_LAYER_SCRATCH,
)


def _tc_layer0(x, p0, p1, w1, b1, w2, b2, gam, bet):
    return pl.pallas_call(
        _layer0_body,
        grid=(2, _NSTEPS),
        in_specs=[_row_block(128), _agg_block(), _agg_block(),
                  _full_block((128, 256)), _full_block((1, 256)),
                  _full_block((256, 256)), _full_block((1, 256)),
                  _full_block((1, 256)), _full_block((1, 256))],
        **_LAYER_OUT,
    )(x, p0, p1, w1, b1, w2, b2, gam, bet)


def _tc_layer12(h0, h1, a0, a1, w1, b1, w2, b2, gam, bet):
    return pl.pallas_call(
        _layer12_body,
        grid=(2, _NSTEPS),
        in_specs=[_row_block(128), _row_block(128), _agg_block(),
                  _agg_block(),
                  _full_block((256, 256)), _full_block((1, 256)),
                  _full_block((256, 256)), _full_block((1, 256)),
                  _full_block((1, 256)), _full_block((1, 256))],
        **_LAYER_OUT,
    )(h0, h1, a0, a1, w1, b1, w2, b2, gam, bet)


def kernel(x, edge_index, batch, W1_0, b1_0, W2_0, b2_0, gam_0, bet_0,
           W1_1, b1_1, W2_1, b2_1, gam_1, bet_1,
           W1_2, b1_2, W2_2, b2_2, gam_2, bet_2, Wm1, bm1, Wm2, bm2):
    src_r = edge_index[0].astype(jnp.int32).reshape(_E // _CH, _CH)
    dst_r = edge_index[1].astype(jnp.int32).reshape(_E // _CH, _CH)
    zblk = jnp.zeros((_RPT_LAST, 128), jnp.float32)

    r = lambda v: v.reshape(1, -1)
    agg_edges = _make_sc_agg(split_edges=True)
    agg_feats = _make_sc_agg(split_edges=False)

    p0, p1 = agg_edges(x, x, src_r, dst_r, zblk)
    h0, h1 = _tc_layer0(x, p0, p1, W1_0, r(b1_0), W2_0, r(b2_0),
                        r(gam_0), r(bet_0))

    a0, a1 = agg_feats(h0, h1, src_r, dst_r, zblk)
    h0, h1 = _tc_layer12(h0, h1, a0, a1, W1_1, r(b1_1), W2_1, r(b2_1),
                         r(gam_1), r(bet_1))

    batch3 = batch.astype(jnp.int32).reshape(_NSTEPS, 1, _BR)
    wm2p = jnp.pad(Wm2, ((0, 0), (0, 118)))
    bm2p = jnp.pad(bm2, (0, 118)).reshape(1, 128)
    a0, a1 = agg_feats(h0, h1, src_r, dst_r, zblk)
    out = _tc_layer2_pool(h0, h1, a0, a1, W1_2, r(b1_2), W2_2, r(b2_2),
                          r(gam_2), r(bet_2), batch3,
                          Wm1, r(bm1), wm2p, bm2p)
    return out[:, :10]


# speedup vs baseline: 9.2558x; 1.0296x over previous
import functools

import jax
import jax.numpy as jnp
from jax import lax
from jax.experimental import pallas as pl
from jax.experimental.pallas import tpu as pltpu
from jax.experimental.pallas import tpu_sc as plsc

_N = 10000
_E = 320000
_G = 64
_NC = 2
_NS = 16
_CH = 125
_RPT = 624
_RPT_LAST = _N - (_NS - 1) * _RPT
_GRP = 16

_BR = 2000
_NSTEPS = _N // _BR


def _make_sc_agg(split_edges: bool):
    if split_edges:
        nch = (_E // _NC) // _NS // _CH
        core_row_off = (_E // _NC) // _CH
        grp = _GRP
    else:
        nch = _E // _NS // _CH
        core_row_off = 0
        grp = 2 * _GRP
    ngrp = nch // grp

    def body(h_a, h_b, src_r, dst_r, zblk, out_a, out_b,
             acc, idxs, idxd, rows0, rows1, sem0, sem1, sem2, sem3, sem4):
        c = lax.axis_index("c")
        s = lax.axis_index("s")

        @pl.when(s < _NS - 1)
        def _():
            pltpu.sync_copy(zblk.at[pl.ds(0, _RPT)],
                            acc.at[pl.ds(s * _RPT, _RPT)])

        @pl.when(s == _NS - 1)
        def _():
            pltpu.sync_copy(zblk, acc.at[pl.ds((_NS - 1) * _RPT, _RPT_LAST)])

        row0 = c * core_row_off + s * nch
        plsc.subcore_barrier()

        def chunk(h):
            rbuf = (rows0, rows1)
            gsem = (sem0, sem1)
            ssem = (sem2, sem3)

            def gather_start(j, b):
                pltpu.async_copy(h.at[idxs.at[j]], rbuf[b], gsem[b])

            def gather_wait(j, b):
                pltpu.make_async_copy(h.at[idxs.at[j]], rbuf[b],
                                      gsem[b]).wait()

            def scatter_start(j, b):
                pltpu.async_copy(rbuf[b], acc.at[idxd.at[j]], ssem[b],
                                 add=True)

            def scatter_wait(j, b):
                pltpu.make_async_copy(rbuf[b], acc.at[idxd.at[j]],
                                      ssem[b]).wait()

            def group(g, carry):
                gr = row0 + g * grp
                pltpu.async_copy(src_r.at[pl.ds(gr, grp)], idxs, sem4)
                pltpu.async_copy(dst_r.at[pl.ds(gr, grp)], idxd, sem4)
                pltpu.make_async_copy(src_r.at[pl.ds(gr, grp)], idxs,
                                      sem4).wait()
                pltpu.make_async_copy(dst_r.at[pl.ds(gr, grp)], idxd,
                                      sem4).wait()
                gather_start(0, 0)
                for j in range(grp):
                    b = j % 2
                    if j + 1 < grp:
                        if j >= 1:
                            scatter_wait(j - 1, 1 - b)
                        gather_start(j + 1, 1 - b)
                    gather_wait(j, b)
                    scatter_start(j, b)
                    if j >= grp - 2:
                        scatter_wait(j, b)
                return carry
            lax.fori_loop(0, ngrp, group, 0)

        @pl.when(c == 0)
        def _():
            chunk(h_a)

        @pl.when(c == 1)
        def _():
            chunk(h_b)

        plsc.subcore_barrier()

        def copy_out(out):
            @pl.when(s < _NS - 1)
            def _():
                pltpu.sync_copy(acc.at[pl.ds(s * _RPT, _RPT)],
                                out.at[pl.ds(s * _RPT, _RPT)])

            @pl.when(s == _NS - 1)
            def _():
                pltpu.sync_copy(acc.at[pl.ds((_NS - 1) * _RPT, _RPT_LAST)],
                                out.at[pl.ds((_NS - 1) * _RPT, _RPT_LAST)])

        @pl.when(c == 0)
        def _():
            copy_out(out_a)

        @pl.when(c == 1)
        def _():
            copy_out(out_b)

    mesh = plsc.VectorSubcoreMesh(core_axis_name="c", subcore_axis_name="s")
    return pl.kernel(
        body,
        out_type=(jax.ShapeDtypeStruct((_N, 128), jnp.float32),
                  jax.ShapeDtypeStruct((_N, 128), jnp.float32)),
        mesh=mesh,
        scratch_types=[
            pltpu.VMEM_SHARED((_N, 128), jnp.float32),
            pltpu.VMEM((grp, _CH), jnp.int32),
            pltpu.VMEM((grp, _CH), jnp.int32),
            pltpu.VMEM((_CH, 128), jnp.float32),
            pltpu.VMEM((_CH, 128), jnp.float32),
            pltpu.SemaphoreType.DMA,
            pltpu.SemaphoreType.DMA,
            pltpu.SemaphoreType.DMA,
            pltpu.SemaphoreType.DMA,
            pltpu.SemaphoreType.DMA,
        ],
    )


def _row_block(din):
    return pl.BlockSpec((_BR, din), lambda p, i: (i, 0))


def _agg_block():
    return pl.BlockSpec((_BR, 128), lambda p, i: (i * (1 - p), 0))


def _full_block(shape):
    return pl.BlockSpec(shape, lambda p, i: tuple(0 for _ in shape))


def _layer_common(i, zp, zscr, s1_ref):
    zscr[pl.ds(i * _BR, _BR), :] = zp
    ss = jnp.concatenate(
        [jnp.sum(zp, axis=0, keepdims=True),
         jnp.sum(zp * zp, axis=0, keepdims=True),
         jnp.zeros((6, zp.shape[1]), jnp.float32)], axis=0)

    @pl.when(i == 0)
    def _():
        s1_ref[...] = ss

    @pl.when(i != 0)
    def _():
        s1_ref[...] = s1_ref[...] + ss


def _layer_finalize(gam_ref, bet_ref, s1_ref, coef_ref):
    inv_n = 1.0 / _N
    mean = s1_ref[0:1, :] * inv_n
    var = s1_ref[1:2, :] * inv_n - mean * mean
    scale = gam_ref[...] / jnp.sqrt(var + 1e-5)
    shift = bet_ref[...] - mean * scale
    coef_ref[...] = jnp.concatenate(
        [scale, shift, jnp.zeros((6, scale.shape[1]), jnp.float32)], axis=0)


def _layer0_body(x_ref, p0_ref, p1_ref, w1_ref, b1_ref, w2_ref, b2_ref,
                 gam_ref, bet_ref, o0_ref, o1_ref,
                 zscr, s1_ref, coef_ref):
    p = pl.program_id(0)
    i = pl.program_id(1)

    @pl.when(p == 0)
    def _():
        zin = x_ref[...] + p0_ref[...] + p1_ref[...]
        z1 = jnp.maximum(jnp.dot(zin, w1_ref[...]) + b1_ref[...], 0.0)
        zp = jnp.dot(z1, w2_ref[...]) + b2_ref[...]
        _layer_common(i, zp, zscr, s1_ref)

    @pl.when(p == 1)
    def _():
        @pl.when(i == 0)
        def _():
            _layer_finalize(gam_ref, bet_ref, s1_ref, coef_ref)
        y = zscr[pl.ds(i * _BR, _BR), :] * coef_ref[0:1, :] \
            + coef_ref[1:2, :]
        y = jnp.maximum(y, 0.0)
        o0_ref[...] = y[:, 0:128]
        o1_ref[...] = y[:, 128:256]


def _layer12_body(h0_ref, h1_ref, a0_ref, a1_ref, w1_ref, b1_ref, w2_ref,
                  b2_ref, gam_ref, bet_ref, o0_ref, o1_ref,
                  zscr, s1_ref, coef_ref):
    p = pl.program_id(0)
    i = pl.program_id(1)

    @pl.when(p == 0)
    def _():
        zin0 = h0_ref[...] + a0_ref[...]
        zin1 = h1_ref[...] + a1_ref[...]
        z1 = jnp.maximum(
            jnp.dot(zin0, w1_ref[0:128, :])
            + jnp.dot(zin1, w1_ref[128:256, :]) + b1_ref[...], 0.0)
        zp = jnp.dot(z1, w2_ref[...]) + b2_ref[...]
        _layer_common(i, zp, zscr, s1_ref)

    @pl.when(p == 1)
    def _():
        @pl.when(i == 0)
        def _():
            _layer_finalize(gam_ref, bet_ref, s1_ref, coef_ref)
        y = zscr[pl.ds(i * _BR, _BR), :] * coef_ref[0:1, :] \
            + coef_ref[1:2, :]
        y = jnp.maximum(y, 0.0)
        o0_ref[...] = y[:, 0:128] + h0_ref[...]
        o1_ref[...] = y[:, 128:256] + h1_ref[...]


def _layer2_pool_body(h0_ref, h1_ref, a0_ref, a1_ref, w1_ref, b1_ref,
                      w2_ref, b2_ref, gam_ref, bet_ref, batch_ref,
                      wm1_ref, bm1_ref, wm2_ref, bm2_ref, out_ref,
                      zscr, s1_ref, coef_ref, sums, counts):
    p = pl.program_id(0)
    i = pl.program_id(1)

    @pl.when(p == 0)
    def _():
        zin0 = h0_ref[...] + a0_ref[...]
        zin1 = h1_ref[...] + a1_ref[...]
        z1 = jnp.maximum(
            jnp.dot(zin0, w1_ref[0:128, :])
            + jnp.dot(zin1, w1_ref[128:256, :]) + b1_ref[...], 0.0)
        zp = jnp.dot(z1, w2_ref[...]) + b2_ref[...]
        _layer_common(i, zp, zscr, s1_ref)

    @pl.when(p == 1)
    def _():
        @pl.when(i == 0)
        def _():
            _layer_finalize(gam_ref, bet_ref, s1_ref, coef_ref)
        y = zscr[pl.ds(i * _BR, _BR), :] * coef_ref[0:1, :] \
            + coef_ref[1:2, :]
        y = jnp.maximum(y, 0.0)
        y0 = y[:, 0:128] + h0_ref[...]
        y1 = y[:, 128:256] + h1_ref[...]
        b = batch_ref[0]
        gids = lax.broadcasted_iota(jnp.int32, (_G, _BR), 0)
        mask = (gids == b).astype(jnp.float32)
        hcat = jnp.concatenate([y0, y1], axis=1)
        part = jnp.dot(mask, hcat)
        cnt = jnp.broadcast_to(jnp.sum(mask, axis=1, keepdims=True),
                               (_G, 128))

        @pl.when(i == 0)
        def _():
            sums[...] = part
            counts[...] = cnt

        @pl.when(i != 0)
        def _():
            sums[...] = sums[...] + part
            counts[...] = counts[...] + cnt

        @pl.when(i == _NSTEPS - 1)
        def _():
            hg = sums[...] / jnp.maximum(counts[:, 0:1], 1.0)
            t = jnp.maximum(jnp.dot(hg, wm1_ref[...]) + bm1_ref[...], 0.0)
            out_ref[...] = jnp.dot(t, wm2_ref[...]) + bm2_ref[...]


_LAYER_SCRATCH = [
    pltpu.VMEM((_N, 256), jnp.float32),
    pltpu.VMEM((8, 256), jnp.float32),
    pltpu.VMEM((8, 256), jnp.float32),
]


def _tc_layer2_pool(h0, h1, a0, a1, w1, b1, w2, b2, gam, bet, batch3,
                    wm1, bm1, wm2p, bm2p):
    return pl.pallas_call(
        _layer2_pool_body,
        grid=(2, _NSTEPS),
        in_specs=[_row_block(128), _row_block(128), _agg_block(),
                  _agg_block(),
                  _full_block((256, 256)), _full_block((1, 256)),
                  _full_block((256, 256)), _full_block((1, 256)),
                  _full_block((1, 256)), _full_block((1, 256)),
                  pl.BlockSpec((1, 1, _BR), lambda p, i: (i * p, 0, 0)),
                  _full_block((256, 256)), _full_block((1, 256)),
                  _full_block((256, 128)), _full_block((1, 128))],
        out_specs=_full_block((_G, 128)),
        out_shape=jax.ShapeDtypeStruct((_G, 128), jnp.float32),
        scratch_shapes=_LAYER_SCRATCH + [
            pltpu.VMEM((_G, 256), jnp.float32),
            pltpu.VMEM((_G, 128), jnp.float32)],
    )(h0, h1, a0, a1, w1, b1, w2, b2, gam, bet, batch3,
      wm1, bm1, wm2p, bm2p)

_LAYER_OUT = dict(
    out_specs=[pl.BlockSpec((_BR, 128), lambda p, i: (i, 0)),
               pl.BlockSpec((_BR, 128), lambda p, i: (i, 0))],
    out_shape=[jax.ShapeDtypeStruct((_N, 128), jnp.float32),
               jax.ShapeDtypeStruct((_N, 128), jnp.float32)],
    scratch_shapes=_LAYER_SCRATCH,
)


def _tc_layer0(x, p0, p1, w1, b1, w2, b2, gam, bet):
    return pl.pallas_call(
        _layer0_body,
        grid=(2, _NSTEPS),
        in_specs=[_row_block(128), _agg_block(), _agg_block(),
                  _full_block((128, 256)), _full_block((1, 256)),
                  _full_block((256, 256)), _full_block((1, 256)),
                  _full_block((1, 256)), _full_block((1, 256))],
        **_LAYER_OUT,
    )(x, p0, p1, w1, b1, w2, b2, gam, bet)


def _tc_layer12(h0, h1, a0, a1, w1, b1, w2, b2, gam, bet):
    return pl.pallas_call(
        _layer12_body,
        grid=(2, _NSTEPS),
        in_specs=[_row_block(128), _row_block(128), _agg_block(),
                  _agg_block(),
                  _full_block((256, 256)), _full_block((1, 256)),
                  _full_block((256, 256)), _full_block((1, 256)),
                  _full_block((1, 256)), _full_block((1, 256))],
        **_LAYER_OUT,
    )(h0, h1, a0, a1, w1, b1, w2, b2, gam, bet)


def kernel(x, edge_index, batch, W1_0, b1_0, W2_0, b2_0, gam_0, bet_0,
           W1_1, b1_1, W2_1, b2_1, gam_1, bet_1,
           W1_2, b1_2, W2_2, b2_2, gam_2, bet_2, Wm1, bm1, Wm2, bm2):
    src_r = edge_index[0].astype(jnp.int32).reshape(_E // _CH, _CH)
    dst_r = edge_index[1].astype(jnp.int32).reshape(_E // _CH, _CH)
    zblk = jnp.zeros((_RPT_LAST, 128), jnp.float32)

    r = lambda v: v.reshape(1, -1)
    agg_edges = _make_sc_agg(split_edges=True)
    agg_feats = _make_sc_agg(split_edges=False)

    p0, p1 = agg_edges(x, x, src_r, dst_r, zblk)
    h0, h1 = _tc_layer0(x, p0, p1, W1_0, r(b1_0), W2_0, r(b2_0),
                        r(gam_0), r(bet_0))

    a0, a1 = agg_feats(h0, h1, src_r, dst_r, zblk)
    h0, h1 = _tc_layer12(h0, h1, a0, a1, W1_1, r(b1_1), W2_1, r(b2_1),
                         r(gam_1), r(bet_1))

    batch3 = batch.astype(jnp.int32).reshape(_NSTEPS, 1, _BR)
    wm2p = jnp.pad(Wm2, ((0, 0), (0, 118)))
    bm2p = jnp.pad(bm2, (0, 118)).reshape(1, 128)
    a0, a1 = agg_feats(h0, h1, src_r, dst_r, zblk)
    out = _tc_layer2_pool(h0, h1, a0, a1, W1_2, r(b1_2), W2_2, r(b2_2),
                          r(gam_2), r(bet_2), batch3,
                          Wm1, r(bm1), wm2p, bm2p)
    return out[:, :10]


# speedup vs baseline: 9.3295x; 1.0080x over previous
import functools

import jax
import jax.numpy as jnp
from jax import lax
from jax.experimental import pallas as pl
from jax.experimental.pallas import tpu as pltpu
from jax.experimental.pallas import tpu_sc as plsc

_N = 10000
_E = 320000
_G = 64
_NC = 2
_NS = 16
_CH = 125
_RPT = 624
_RPT_LAST = _N - (_NS - 1) * _RPT
_GRP = 16

_BR = 2000
_NSTEPS = _N // _BR


def _make_sc_agg(split_edges: bool):
    if split_edges:
        nch = (_E // _NC) // _NS // _CH
        core_row_off = (_E // _NC) // _CH
        grp = _GRP
    else:
        nch = _E // _NS // _CH
        core_row_off = 0
        grp = 2 * _GRP
    ngrp = nch // grp

    def body(h_a, h_b, src_r, dst_r, zblk, out_a, out_b,
             acc, idxs, idxd, rows0, rows1, sem0, sem1, sem2, sem3, sem4):
        c = lax.axis_index("c")
        s = lax.axis_index("s")

        @pl.when(s < _NS - 1)
        def _():
            pltpu.sync_copy(zblk.at[pl.ds(0, _RPT)],
                            acc.at[pl.ds(s * _RPT, _RPT)])

        @pl.when(s == _NS - 1)
        def _():
            pltpu.sync_copy(zblk, acc.at[pl.ds((_NS - 1) * _RPT, _RPT_LAST)])

        row0 = c * core_row_off + s * nch
        plsc.subcore_barrier()

        def chunk(h):
            rbuf = (rows0, rows1)
            gsem = (sem0, sem1)
            ssem = (sem2, sem3)

            def gather_start(j, b):
                pltpu.async_copy(h.at[idxs.at[j]], rbuf[b], gsem[b])

            def gather_wait(j, b):
                pltpu.make_async_copy(h.at[idxs.at[j]], rbuf[b],
                                      gsem[b]).wait()

            def scatter_start(j, b):
                pltpu.async_copy(rbuf[b], acc.at[idxd.at[j]], ssem[b],
                                 add=True)

            def scatter_wait(j, b):
                pltpu.make_async_copy(rbuf[b], acc.at[idxd.at[j]],
                                      ssem[b]).wait()

            def group(g, carry):
                gr = row0 + g * grp
                pltpu.async_copy(src_r.at[pl.ds(gr, grp)], idxs, sem4)
                pltpu.async_copy(dst_r.at[pl.ds(gr, grp)], idxd, sem4)
                pltpu.make_async_copy(src_r.at[pl.ds(gr, grp)], idxs,
                                      sem4).wait()
                pltpu.make_async_copy(dst_r.at[pl.ds(gr, grp)], idxd,
                                      sem4).wait()
                gather_start(0, 0)
                for j in range(grp):
                    b = j % 2
                    if j + 1 < grp:
                        if j >= 1:
                            scatter_wait(j - 1, 1 - b)
                        gather_start(j + 1, 1 - b)
                    gather_wait(j, b)
                    scatter_start(j, b)
                    if j >= grp - 2:
                        scatter_wait(j, b)
                return carry
            lax.fori_loop(0, ngrp, group, 0)

        @pl.when(c == 0)
        def _():
            chunk(h_a)

        @pl.when(c == 1)
        def _():
            chunk(h_b)

        plsc.subcore_barrier()

        def copy_out(out):
            @pl.when(s < _NS - 1)
            def _():
                pltpu.sync_copy(acc.at[pl.ds(s * _RPT, _RPT)],
                                out.at[pl.ds(s * _RPT, _RPT)])

            @pl.when(s == _NS - 1)
            def _():
                pltpu.sync_copy(acc.at[pl.ds((_NS - 1) * _RPT, _RPT_LAST)],
                                out.at[pl.ds((_NS - 1) * _RPT, _RPT_LAST)])

        @pl.when(c == 0)
        def _():
            copy_out(out_a)

        @pl.when(c == 1)
        def _():
            copy_out(out_b)

    mesh = plsc.VectorSubcoreMesh(core_axis_name="c", subcore_axis_name="s")
    return pl.kernel(
        body,
        out_type=(jax.ShapeDtypeStruct((_N, 128), jnp.float32),
                  jax.ShapeDtypeStruct((_N, 128), jnp.float32)),
        mesh=mesh,
        scratch_types=[
            pltpu.VMEM_SHARED((_N, 128), jnp.float32),
            pltpu.VMEM((grp, _CH), jnp.int32),
            pltpu.VMEM((grp, _CH), jnp.int32),
            pltpu.VMEM((_CH, 128), jnp.float32),
            pltpu.VMEM((_CH, 128), jnp.float32),
            pltpu.SemaphoreType.DMA,
            pltpu.SemaphoreType.DMA,
            pltpu.SemaphoreType.DMA,
            pltpu.SemaphoreType.DMA,
            pltpu.SemaphoreType.DMA,
        ],
    )


def _row_block(din):
    return pl.BlockSpec((_BR, din), lambda p, i: (i, 0))


def _agg_block():
    return pl.BlockSpec((_BR, 128), lambda p, i: (i * (1 - p), 0))


def _full_block(shape):
    return pl.BlockSpec(shape, lambda p, i: tuple(0 for _ in shape))


def _layer_common(i, zp, zscr, s1_ref):
    zscr[pl.ds(i * _BR, _BR), :] = zp
    ss = jnp.concatenate(
        [jnp.sum(zp, axis=0, keepdims=True),
         jnp.sum(zp * zp, axis=0, keepdims=True),
         jnp.zeros((6, zp.shape[1]), jnp.float32)], axis=0)

    @pl.when(i == 0)
    def _():
        s1_ref[...] = ss

    @pl.when(i != 0)
    def _():
        s1_ref[...] = s1_ref[...] + ss


def _layer_finalize(gam_ref, bet_ref, s1_ref, coef_ref):
    inv_n = 1.0 / _N
    mean = s1_ref[0:1, :] * inv_n
    var = s1_ref[1:2, :] * inv_n - mean * mean
    scale = gam_ref[...] / jnp.sqrt(var + 1e-5)
    shift = bet_ref[...] - mean * scale
    coef_ref[...] = jnp.concatenate(
        [scale, shift, jnp.zeros((6, scale.shape[1]), jnp.float32)], axis=0)


def _layer0_body(x_ref, p0_ref, p1_ref, w1_ref, b1_ref, w2_ref, b2_ref,
                 gam_ref, bet_ref, o0_ref, o1_ref,
                 zscr, s1_ref, coef_ref):
    p = pl.program_id(0)
    i = pl.program_id(1)

    @pl.when(p == 0)
    def _():
        zin = x_ref[...] + p0_ref[...] + p1_ref[...]
        z1 = jnp.maximum(jnp.dot(zin, w1_ref[...]) + b1_ref[...], 0.0)
        zp = jnp.dot(z1, w2_ref[...]) + b2_ref[...]
        _layer_common(i, zp, zscr, s1_ref)

    @pl.when(p == 1)
    def _():
        @pl.when(i == 0)
        def _():
            _layer_finalize(gam_ref, bet_ref, s1_ref, coef_ref)
        y = zscr[pl.ds(i * _BR, _BR), :] * coef_ref[0:1, :] \
            + coef_ref[1:2, :]
        y = jnp.maximum(y, 0.0)
        o0_ref[...] = y[:, 0:128]
        o1_ref[...] = y[:, 128:256]


def _layer12_body(h0_ref, h1_ref, a0_ref, a1_ref, w1_ref, b1_ref, w2_ref,
                  b2_ref, gam_ref, bet_ref, o0_ref, o1_ref,
                  zscr, s1_ref, coef_ref, hscr0, hscr1):
    p = pl.program_id(0)
    i = pl.program_id(1)

    @pl.when(p == 0)
    def _():
        h0b = h0_ref[...]
        h1b = h1_ref[...]
        hscr0[pl.ds(i * _BR, _BR), :] = h0b
        hscr1[pl.ds(i * _BR, _BR), :] = h1b
        zin0 = h0b + a0_ref[...]
        zin1 = h1b + a1_ref[...]
        z1 = jnp.maximum(
            jnp.dot(zin0, w1_ref[0:128, :])
            + jnp.dot(zin1, w1_ref[128:256, :]) + b1_ref[...], 0.0)
        zp = jnp.dot(z1, w2_ref[...]) + b2_ref[...]
        _layer_common(i, zp, zscr, s1_ref)

    @pl.when(p == 1)
    def _():
        @pl.when(i == 0)
        def _():
            _layer_finalize(gam_ref, bet_ref, s1_ref, coef_ref)
        y = zscr[pl.ds(i * _BR, _BR), :] * coef_ref[0:1, :] \
            + coef_ref[1:2, :]
        y = jnp.maximum(y, 0.0)
        o0_ref[...] = y[:, 0:128] + hscr0[pl.ds(i * _BR, _BR), :]
        o1_ref[...] = y[:, 128:256] + hscr1[pl.ds(i * _BR, _BR), :]


def _layer2_pool_body(h0_ref, h1_ref, a0_ref, a1_ref, w1_ref, b1_ref,
                      w2_ref, b2_ref, gam_ref, bet_ref, batch_ref,
                      wm1_ref, bm1_ref, wm2_ref, bm2_ref, out_ref,
                      zscr, s1_ref, coef_ref, sums, counts, hscr0, hscr1):
    p = pl.program_id(0)
    i = pl.program_id(1)

    @pl.when(p == 0)
    def _():
        h0b = h0_ref[...]
        h1b = h1_ref[...]
        hscr0[pl.ds(i * _BR, _BR), :] = h0b
        hscr1[pl.ds(i * _BR, _BR), :] = h1b
        zin0 = h0b + a0_ref[...]
        zin1 = h1b + a1_ref[...]
        z1 = jnp.maximum(
            jnp.dot(zin0, w1_ref[0:128, :])
            + jnp.dot(zin1, w1_ref[128:256, :]) + b1_ref[...], 0.0)
        zp = jnp.dot(z1, w2_ref[...]) + b2_ref[...]
        _layer_common(i, zp, zscr, s1_ref)

    @pl.when(p == 1)
    def _():
        @pl.when(i == 0)
        def _():
            _layer_finalize(gam_ref, bet_ref, s1_ref, coef_ref)
        y = zscr[pl.ds(i * _BR, _BR), :] * coef_ref[0:1, :] \
            + coef_ref[1:2, :]
        y = jnp.maximum(y, 0.0)
        y0 = y[:, 0:128] + hscr0[pl.ds(i * _BR, _BR), :]
        y1 = y[:, 128:256] + hscr1[pl.ds(i * _BR, _BR), :]
        b = batch_ref[0]
        gids = lax.broadcasted_iota(jnp.int32, (_G, _BR), 0)
        mask = (gids == b).astype(jnp.float32)
        hcat = jnp.concatenate([y0, y1], axis=1)
        part = jnp.dot(mask, hcat)
        cnt = jnp.broadcast_to(jnp.sum(mask, axis=1, keepdims=True),
                               (_G, 128))

        @pl.when(i == 0)
        def _():
            sums[...] = part
            counts[...] = cnt

        @pl.when(i != 0)
        def _():
            sums[...] = sums[...] + part
            counts[...] = counts[...] + cnt

        @pl.when(i == _NSTEPS - 1)
        def _():
            hg = sums[...] / jnp.maximum(counts[:, 0:1], 1.0)
            t = jnp.maximum(jnp.dot(hg, wm1_ref[...]) + bm1_ref[...], 0.0)
            out_ref[...] = jnp.dot(t, wm2_ref[...]) + bm2_ref[...]


_LAYER_SCRATCH = [
    pltpu.VMEM((_N, 256), jnp.float32),
    pltpu.VMEM((8, 256), jnp.float32),
    pltpu.VMEM((8, 256), jnp.float32),
]

_HRES_SCRATCH = [
    pltpu.VMEM((_N, 128), jnp.float32),
    pltpu.VMEM((_N, 128), jnp.float32),
]


def _tc_layer2_pool(h0, h1, a0, a1, w1, b1, w2, b2, gam, bet, batch3,
                    wm1, bm1, wm2p, bm2p):
    return pl.pallas_call(
        _layer2_pool_body,
        grid=(2, _NSTEPS),
        in_specs=[_agg_block(), _agg_block(), _agg_block(),
                  _agg_block(),
                  _full_block((256, 256)), _full_block((1, 256)),
                  _full_block((256, 256)), _full_block((1, 256)),
                  _full_block((1, 256)), _full_block((1, 256)),
                  pl.BlockSpec((1, 1, _BR), lambda p, i: (i * p, 0, 0)),
                  _full_block((256, 256)), _full_block((1, 256)),
                  _full_block((256, 128)), _full_block((1, 128))],
        out_specs=_full_block((_G, 128)),
        out_shape=jax.ShapeDtypeStruct((_G, 128), jnp.float32),
        scratch_shapes=_LAYER_SCRATCH + [
            pltpu.VMEM((_G, 256), jnp.float32),
            pltpu.VMEM((_G, 128), jnp.float32)] + _HRES_SCRATCH,
    )(h0, h1, a0, a1, w1, b1, w2, b2, gam, bet, batch3,
      wm1, bm1, wm2p, bm2p)

_LAYER_OUT = dict(
    out_specs=[pl.BlockSpec((_BR, 128), lambda p, i: (i, 0)),
               pl.BlockSpec((_BR, 128), lambda p, i: (i, 0))],
    out_shape=[jax.ShapeDtypeStruct((_N, 128), jnp.float32),
               jax.ShapeDtypeStruct((_N, 128), jnp.float32)],
    scratch_shapes=_LAYER_SCRATCH,
)


def _tc_layer0(x, p0, p1, w1, b1, w2, b2, gam, bet):
    return pl.pallas_call(
        _layer0_body,
        grid=(2, _NSTEPS),
        in_specs=[_row_block(128), _agg_block(), _agg_block(),
                  _full_block((128, 256)), _full_block((1, 256)),
                  _full_block((256, 256)), _full_block((1, 256)),
                  _full_block((1, 256)), _full_block((1, 256))],
        **_LAYER_OUT,
    )(x, p0, p1, w1, b1, w2, b2, gam, bet)


def _tc_layer12(h0, h1, a0, a1, w1, b1, w2, b2, gam, bet):
    return pl.pallas_call(
        _layer12_body,
        grid=(2, _NSTEPS),
        in_specs=[_agg_block(), _agg_block(), _agg_block(),
                  _agg_block(),
                  _full_block((256, 256)), _full_block((1, 256)),
                  _full_block((256, 256)), _full_block((1, 256)),
                  _full_block((1, 256)), _full_block((1, 256))],
        out_specs=_LAYER_OUT["out_specs"],
        out_shape=_LAYER_OUT["out_shape"],
        scratch_shapes=_LAYER_SCRATCH + _HRES_SCRATCH,
    )(h0, h1, a0, a1, w1, b1, w2, b2, gam, bet)


def kernel(x, edge_index, batch, W1_0, b1_0, W2_0, b2_0, gam_0, bet_0,
           W1_1, b1_1, W2_1, b2_1, gam_1, bet_1,
           W1_2, b1_2, W2_2, b2_2, gam_2, bet_2, Wm1, bm1, Wm2, bm2):
    src_r = edge_index[0].astype(jnp.int32).reshape(_E // _CH, _CH)
    dst_r = edge_index[1].astype(jnp.int32).reshape(_E // _CH, _CH)
    zblk = jnp.zeros((_RPT_LAST, 128), jnp.float32)

    r = lambda v: v.reshape(1, -1)
    agg_edges = _make_sc_agg(split_edges=True)
    agg_feats = _make_sc_agg(split_edges=False)

    p0, p1 = agg_edges(x, x, src_r, dst_r, zblk)
    h0, h1 = _tc_layer0(x, p0, p1, W1_0, r(b1_0), W2_0, r(b2_0),
                        r(gam_0), r(bet_0))

    a0, a1 = agg_feats(h0, h1, src_r, dst_r, zblk)
    h0, h1 = _tc_layer12(h0, h1, a0, a1, W1_1, r(b1_1), W2_1, r(b2_1),
                         r(gam_1), r(bet_1))

    batch3 = batch.astype(jnp.int32).reshape(_NSTEPS, 1, _BR)
    wm2p = jnp.pad(Wm2, ((0, 0), (0, 118)))
    bm2p = jnp.pad(bm2, (0, 118)).reshape(1, 128)
    a0, a1 = agg_feats(h0, h1, src_r, dst_r, zblk)
    out = _tc_layer2_pool(h0, h1, a0, a1, W1_2, r(b1_2), W2_2, r(b2_2),
                          r(gam_2), r(bet_2), batch3,
                          Wm1, r(bm1), wm2p, bm2p)
    return out[:, :10]


# speedup vs baseline: 9.3525x; 1.0025x over previous
import functools

import jax
import jax.numpy as jnp
from jax import lax
from jax.experimental import pallas as pl
from jax.experimental.pallas import tpu as pltpu
from jax.experimental.pallas import tpu_sc as plsc

_N = 10000
_E = 320000
_G = 64
_NC = 2
_NS = 16
_CH = 125
_RPT = 624
_RPT_LAST = _N - (_NS - 1) * _RPT
_GRP = 16

_BR = 2000
_NSTEPS = _N // _BR


def _make_sc_agg(split_edges: bool):
    if split_edges:
        nch = (_E // _NC) // _NS // _CH
        core_row_off = (_E // _NC) // _CH
        grp = _GRP
    else:
        nch = _E // _NS // _CH
        core_row_off = 0
        grp = 2 * _GRP
    ngrp = nch // grp

    def body(h_a, h_b, src_r, dst_r, zblk, out_a, out_b,
             acc, idxs, idxd, rows0, rows1,
             sem0, sem1, sem2, sem3, sem4, sem5):
        c = lax.axis_index("c")
        s = lax.axis_index("s")

        @pl.when(s < _NS - 1)
        def _():
            pltpu.async_copy(zblk.at[pl.ds(0, _RPT)],
                             acc.at[pl.ds(s * _RPT, _RPT)], sem5)

        @pl.when(s == _NS - 1)
        def _():
            pltpu.async_copy(zblk, acc.at[pl.ds((_NS - 1) * _RPT, _RPT_LAST)],
                             sem5)

        row0 = c * core_row_off + s * nch
        pltpu.async_copy(src_r.at[pl.ds(row0, grp)], idxs, sem4)
        pltpu.async_copy(dst_r.at[pl.ds(row0, grp)], idxd, sem4)
        pltpu.make_async_copy(src_r.at[pl.ds(row0, grp)], idxs, sem4).wait()
        pltpu.make_async_copy(dst_r.at[pl.ds(row0, grp)], idxd, sem4).wait()

        @pl.when(s < _NS - 1)
        def _():
            pltpu.make_async_copy(zblk.at[pl.ds(0, _RPT)],
                                  acc.at[pl.ds(s * _RPT, _RPT)], sem5).wait()

        @pl.when(s == _NS - 1)
        def _():
            pltpu.make_async_copy(zblk,
                                  acc.at[pl.ds((_NS - 1) * _RPT, _RPT_LAST)],
                                  sem5).wait()

        plsc.subcore_barrier()

        def chunk(h):
            rbuf = (rows0, rows1)
            gsem = (sem0, sem1)
            ssem = (sem2, sem3)

            def gather_start(j, b):
                pltpu.async_copy(h.at[idxs.at[j]], rbuf[b], gsem[b])

            def gather_wait(j, b):
                pltpu.make_async_copy(h.at[idxs.at[j]], rbuf[b],
                                      gsem[b]).wait()

            def scatter_start(j, b):
                pltpu.async_copy(rbuf[b], acc.at[idxd.at[j]], ssem[b],
                                 add=True)

            def scatter_wait(j, b):
                pltpu.make_async_copy(rbuf[b], acc.at[idxd.at[j]],
                                      ssem[b]).wait()

            def group(g, carry):
                gr = row0 + g * grp

                @pl.when(g > 0)
                def _():
                    pltpu.async_copy(src_r.at[pl.ds(gr, grp)], idxs, sem4)
                    pltpu.async_copy(dst_r.at[pl.ds(gr, grp)], idxd, sem4)
                    pltpu.make_async_copy(src_r.at[pl.ds(gr, grp)], idxs,
                                          sem4).wait()
                    pltpu.make_async_copy(dst_r.at[pl.ds(gr, grp)], idxd,
                                          sem4).wait()
                gather_start(0, 0)
                for j in range(grp):
                    b = j % 2
                    if j + 1 < grp:
                        if j >= 1:
                            scatter_wait(j - 1, 1 - b)
                        gather_start(j + 1, 1 - b)
                    gather_wait(j, b)
                    scatter_start(j, b)
                    if j >= grp - 2:
                        scatter_wait(j, b)
                return carry
            lax.fori_loop(0, ngrp, group, 0)

        @pl.when(c == 0)
        def _():
            chunk(h_a)

        @pl.when(c == 1)
        def _():
            chunk(h_b)

        plsc.subcore_barrier()

        def copy_out(out):
            @pl.when(s < _NS - 1)
            def _():
                pltpu.sync_copy(acc.at[pl.ds(s * _RPT, _RPT)],
                                out.at[pl.ds(s * _RPT, _RPT)])

            @pl.when(s == _NS - 1)
            def _():
                pltpu.sync_copy(acc.at[pl.ds((_NS - 1) * _RPT, _RPT_LAST)],
                                out.at[pl.ds((_NS - 1) * _RPT, _RPT_LAST)])

        @pl.when(c == 0)
        def _():
            copy_out(out_a)

        @pl.when(c == 1)
        def _():
            copy_out(out_b)

    mesh = plsc.VectorSubcoreMesh(core_axis_name="c", subcore_axis_name="s")
    return pl.kernel(
        body,
        out_type=(jax.ShapeDtypeStruct((_N, 128), jnp.float32),
                  jax.ShapeDtypeStruct((_N, 128), jnp.float32)),
        mesh=mesh,
        scratch_types=[
            pltpu.VMEM_SHARED((_N, 128), jnp.float32),
            pltpu.VMEM((grp, _CH), jnp.int32),
            pltpu.VMEM((grp, _CH), jnp.int32),
            pltpu.VMEM((_CH, 128), jnp.float32),
            pltpu.VMEM((_CH, 128), jnp.float32),
            pltpu.SemaphoreType.DMA,
            pltpu.SemaphoreType.DMA,
            pltpu.SemaphoreType.DMA,
            pltpu.SemaphoreType.DMA,
            pltpu.SemaphoreType.DMA,
            pltpu.SemaphoreType.DMA,
        ],
    )


def _row_block(din):
    return pl.BlockSpec((_BR, din), lambda p, i: (i, 0))


def _agg_block():
    return pl.BlockSpec((_BR, 128), lambda p, i: (i * (1 - p), 0))


def _full_block(shape):
    return pl.BlockSpec(shape, lambda p, i: tuple(0 for _ in shape))


def _layer_common(i, zp, zscr, s1_ref):
    zscr[pl.ds(i * _BR, _BR), :] = zp
    ss = jnp.concatenate(
        [jnp.sum(zp, axis=0, keepdims=True),
         jnp.sum(zp * zp, axis=0, keepdims=True),
         jnp.zeros((6, zp.shape[1]), jnp.float32)], axis=0)

    @pl.when(i == 0)
    def _():
        s1_ref[...] = ss

    @pl.when(i != 0)
    def _():
        s1_ref[...] = s1_ref[...] + ss


def _layer_finalize(gam_ref, bet_ref, s1_ref, coef_ref):
    inv_n = 1.0 / _N
    mean = s1_ref[0:1, :] * inv_n
    var = s1_ref[1:2, :] * inv_n - mean * mean
    scale = gam_ref[...] / jnp.sqrt(var + 1e-5)
    shift = bet_ref[...] - mean * scale
    coef_ref[...] = jnp.concatenate(
        [scale, shift, jnp.zeros((6, scale.shape[1]), jnp.float32)], axis=0)


def _layer0_body(x_ref, p0_ref, p1_ref, w1_ref, b1_ref, w2_ref, b2_ref,
                 gam_ref, bet_ref, o0_ref, o1_ref,
                 zscr, s1_ref, coef_ref):
    p = pl.program_id(0)
    i = pl.program_id(1)

    @pl.when(p == 0)
    def _():
        zin = x_ref[...] + p0_ref[...] + p1_ref[...]
        z1 = jnp.maximum(jnp.dot(zin, w1_ref[...]) + b1_ref[...], 0.0)
        zp = jnp.dot(z1, w2_ref[...]) + b2_ref[...]
        _layer_common(i, zp, zscr, s1_ref)

    @pl.when(p == 1)
    def _():
        @pl.when(i == 0)
        def _():
            _layer_finalize(gam_ref, bet_ref, s1_ref, coef_ref)
        y = zscr[pl.ds(i * _BR, _BR), :] * coef_ref[0:1, :] \
            + coef_ref[1:2, :]
        y = jnp.maximum(y, 0.0)
        o0_ref[...] = y[:, 0:128]
        o1_ref[...] = y[:, 128:256]


def _layer12_body(h0_ref, h1_ref, a0_ref, a1_ref, w1_ref, b1_ref, w2_ref,
                  b2_ref, gam_ref, bet_ref, o0_ref, o1_ref,
                  zscr, s1_ref, coef_ref, hscr0, hscr1):
    p = pl.program_id(0)
    i = pl.program_id(1)

    @pl.when(p == 0)
    def _():
        h0b = h0_ref[...]
        h1b = h1_ref[...]
        hscr0[pl.ds(i * _BR, _BR), :] = h0b
        hscr1[pl.ds(i * _BR, _BR), :] = h1b
        zin0 = h0b + a0_ref[...]
        zin1 = h1b + a1_ref[...]
        z1 = jnp.maximum(
            jnp.dot(zin0, w1_ref[0:128, :])
            + jnp.dot(zin1, w1_ref[128:256, :]) + b1_ref[...], 0.0)
        zp = jnp.dot(z1, w2_ref[...]) + b2_ref[...]
        _layer_common(i, zp, zscr, s1_ref)

    @pl.when(p == 1)
    def _():
        @pl.when(i == 0)
        def _():
            _layer_finalize(gam_ref, bet_ref, s1_ref, coef_ref)
        y = zscr[pl.ds(i * _BR, _BR), :] * coef_ref[0:1, :] \
            + coef_ref[1:2, :]
        y = jnp.maximum(y, 0.0)
        o0_ref[...] = y[:, 0:128] + hscr0[pl.ds(i * _BR, _BR), :]
        o1_ref[...] = y[:, 128:256] + hscr1[pl.ds(i * _BR, _BR), :]


def _layer2_pool_body(h0_ref, h1_ref, a0_ref, a1_ref, w1_ref, b1_ref,
                      w2_ref, b2_ref, gam_ref, bet_ref, batch_ref,
                      wm1_ref, bm1_ref, wm2_ref, bm2_ref, out_ref,
                      zscr, s1_ref, coef_ref, sums, counts, hscr0, hscr1):
    p = pl.program_id(0)
    i = pl.program_id(1)

    @pl.when(p == 0)
    def _():
        h0b = h0_ref[...]
        h1b = h1_ref[...]
        hscr0[pl.ds(i * _BR, _BR), :] = h0b
        hscr1[pl.ds(i * _BR, _BR), :] = h1b
        zin0 = h0b + a0_ref[...]
        zin1 = h1b + a1_ref[...]
        z1 = jnp.maximum(
            jnp.dot(zin0, w1_ref[0:128, :])
            + jnp.dot(zin1, w1_ref[128:256, :]) + b1_ref[...], 0.0)
        zp = jnp.dot(z1, w2_ref[...]) + b2_ref[...]
        _layer_common(i, zp, zscr, s1_ref)

    @pl.when(p == 1)
    def _():
        @pl.when(i == 0)
        def _():
            _layer_finalize(gam_ref, bet_ref, s1_ref, coef_ref)
        y = zscr[pl.ds(i * _BR, _BR), :] * coef_ref[0:1, :] \
            + coef_ref[1:2, :]
        y = jnp.maximum(y, 0.0)
        y0 = y[:, 0:128] + hscr0[pl.ds(i * _BR, _BR), :]
        y1 = y[:, 128:256] + hscr1[pl.ds(i * _BR, _BR), :]
        b = batch_ref[0]
        gids = lax.broadcasted_iota(jnp.int32, (_G, _BR), 0)
        mask = (gids == b).astype(jnp.float32)
        hcat = jnp.concatenate([y0, y1], axis=1)
        part = jnp.dot(mask, hcat)
        cnt = jnp.broadcast_to(jnp.sum(mask, axis=1, keepdims=True),
                               (_G, 128))

        @pl.when(i == 0)
        def _():
            sums[...] = part
            counts[...] = cnt

        @pl.when(i != 0)
        def _():
            sums[...] = sums[...] + part
            counts[...] = counts[...] + cnt

        @pl.when(i == _NSTEPS - 1)
        def _():
            hg = sums[...] / jnp.maximum(counts[:, 0:1], 1.0)
            t = jnp.maximum(jnp.dot(hg, wm1_ref[...]) + bm1_ref[...], 0.0)
            out_ref[...] = jnp.dot(t, wm2_ref[...]) + bm2_ref[...]


_LAYER_SCRATCH = [
    pltpu.VMEM((_N, 256), jnp.float32),
    pltpu.VMEM((8, 256), jnp.float32),
    pltpu.VMEM((8, 256), jnp.float32),
]

_HRES_SCRATCH = [
    pltpu.VMEM((_N, 128), jnp.float32),
    pltpu.VMEM((_N, 128), jnp.float32),
]


def _tc_layer2_pool(h0, h1, a0, a1, w1, b1, w2, b2, gam, bet, batch3,
                    wm1, bm1, wm2p, bm2p):
    return pl.pallas_call(
        _layer2_pool_body,
        grid=(2, _NSTEPS),
        in_specs=[_agg_block(), _agg_block(), _agg_block(),
                  _agg_block(),
                  _full_block((256, 256)), _full_block((1, 256)),
                  _full_block((256, 256)), _full_block((1, 256)),
                  _full_block((1, 256)), _full_block((1, 256)),
                  pl.BlockSpec((1, 1, _BR), lambda p, i: (i * p, 0, 0)),
                  _full_block((256, 256)), _full_block((1, 256)),
                  _full_block((256, 128)), _full_block((1, 128))],
        out_specs=_full_block((_G, 128)),
        out_shape=jax.ShapeDtypeStruct((_G, 128), jnp.float32),
        scratch_shapes=_LAYER_SCRATCH + [
            pltpu.VMEM((_G, 256), jnp.float32),
            pltpu.VMEM((_G, 128), jnp.float32)] + _HRES_SCRATCH,
    )(h0, h1, a0, a1, w1, b1, w2, b2, gam, bet, batch3,
      wm1, bm1, wm2p, bm2p)

_LAYER_OUT = dict(
    out_specs=[pl.BlockSpec((_BR, 128), lambda p, i: (i, 0)),
               pl.BlockSpec((_BR, 128), lambda p, i: (i, 0))],
    out_shape=[jax.ShapeDtypeStruct((_N, 128), jnp.float32),
               jax.ShapeDtypeStruct((_N, 128), jnp.float32)],
    scratch_shapes=_LAYER_SCRATCH,
)


def _tc_layer0(x, p0, p1, w1, b1, w2, b2, gam, bet):
    return pl.pallas_call(
        _layer0_body,
        grid=(2, _NSTEPS),
        in_specs=[_row_block(128), _agg_block(), _agg_block(),
                  _full_block((128, 256)), _full_block((1, 256)),
                  _full_block((256, 256)), _full_block((1, 256)),
                  _full_block((1, 256)), _full_block((1, 256))],
        **_LAYER_OUT,
    )(x, p0, p1, w1, b1, w2, b2, gam, bet)


def _tc_layer12(h0, h1, a0, a1, w1, b1, w2, b2, gam, bet):
    return pl.pallas_call(
        _layer12_body,
        grid=(2, _NSTEPS),
        in_specs=[_agg_block(), _agg_block(), _agg_block(),
                  _agg_block(),
                  _full_block((256, 256)), _full_block((1, 256)),
                  _full_block((256, 256)), _full_block((1, 256)),
                  _full_block((1, 256)), _full_block((1, 256))],
        out_specs=_LAYER_OUT["out_specs"],
        out_shape=_LAYER_OUT["out_shape"],
        scratch_shapes=_LAYER_SCRATCH + _HRES_SCRATCH,
    )(h0, h1, a0, a1, w1, b1, w2, b2, gam, bet)


def kernel(x, edge_index, batch, W1_0, b1_0, W2_0, b2_0, gam_0, bet_0,
           W1_1, b1_1, W2_1, b2_1, gam_1, bet_1,
           W1_2, b1_2, W2_2, b2_2, gam_2, bet_2, Wm1, bm1, Wm2, bm2):
    src_r = edge_index[0].astype(jnp.int32).reshape(_E // _CH, _CH)
    dst_r = edge_index[1].astype(jnp.int32).reshape(_E // _CH, _CH)
    zblk = jnp.zeros((_RPT_LAST, 128), jnp.float32)

    r = lambda v: v.reshape(1, -1)
    agg_edges = _make_sc_agg(split_edges=True)
    agg_feats = _make_sc_agg(split_edges=False)

    p0, p1 = agg_edges(x, x, src_r, dst_r, zblk)
    h0, h1 = _tc_layer0(x, p0, p1, W1_0, r(b1_0), W2_0, r(b2_0),
                        r(gam_0), r(bet_0))

    a0, a1 = agg_feats(h0, h1, src_r, dst_r, zblk)
    h0, h1 = _tc_layer12(h0, h1, a0, a1, W1_1, r(b1_1), W2_1, r(b2_1),
                         r(gam_1), r(bet_1))

    batch3 = batch.astype(jnp.int32).reshape(_NSTEPS, 1, _BR)
    wm2p = jnp.pad(Wm2, ((0, 0), (0, 118)))
    bm2p = jnp.pad(bm2, (0, 118)).reshape(1, 128)
    a0, a1 = agg_feats(h0, h1, src_r, dst_r, zblk)
    out = _tc_layer2_pool(h0, h1, a0, a1, W1_2, r(b1_2), W2_2, r(b2_2),
                          r(gam_2), r(bet_2), batch3,
                          Wm1, r(bm1), wm2p, bm2p)
    return out[:, :10]


# speedup vs baseline: 9.4401x; 1.0094x over previous
import functools

import jax
import jax.numpy as jnp
from jax import lax
from jax.experimental import pallas as pl
from jax.experimental.pallas import tpu as pltpu
from jax.experimental.pallas import tpu_sc as plsc

_N = 10000
_E = 320000
_G = 64
_NC = 2
_NS = 16
_CH = 125
_RPT = 624
_RPT_LAST = _N - (_NS - 1) * _RPT
_GRP = 16

_BR = 2000
_NSTEPS = _N // _BR


def _make_sc_agg(split_edges: bool):
    if split_edges:
        nch = (_E // _NC) // _NS // _CH
        core_row_off = (_E // _NC) // _CH
        grp = 40
    else:
        nch = _E // _NS // _CH
        core_row_off = 0
        grp = 2 * _GRP
    ngrp = nch // grp

    def body(h_a, h_b, src_r, dst_r, zblk, out_a, out_b,
             acc, idxs, idxd, rows0, rows1,
             sem0, sem1, sem2, sem3, sem4, sem5):
        c = lax.axis_index("c")
        s = lax.axis_index("s")

        @pl.when(s < _NS - 1)
        def _():
            pltpu.async_copy(zblk.at[pl.ds(0, _RPT)],
                             acc.at[pl.ds(s * _RPT, _RPT)], sem5)

        @pl.when(s == _NS - 1)
        def _():
            pltpu.async_copy(zblk, acc.at[pl.ds((_NS - 1) * _RPT, _RPT_LAST)],
                             sem5)

        row0 = c * core_row_off + s * nch
        pltpu.async_copy(src_r.at[pl.ds(row0, grp)], idxs, sem4)
        pltpu.async_copy(dst_r.at[pl.ds(row0, grp)], idxd, sem4)
        pltpu.make_async_copy(src_r.at[pl.ds(row0, grp)], idxs, sem4).wait()
        pltpu.make_async_copy(dst_r.at[pl.ds(row0, grp)], idxd, sem4).wait()

        @pl.when(s < _NS - 1)
        def _():
            pltpu.make_async_copy(zblk.at[pl.ds(0, _RPT)],
                                  acc.at[pl.ds(s * _RPT, _RPT)], sem5).wait()

        @pl.when(s == _NS - 1)
        def _():
            pltpu.make_async_copy(zblk,
                                  acc.at[pl.ds((_NS - 1) * _RPT, _RPT_LAST)],
                                  sem5).wait()

        plsc.subcore_barrier()

        def chunk(h):
            rbuf = (rows0, rows1)
            gsem = (sem0, sem1)
            ssem = (sem2, sem3)

            def gather_start(j, b):
                pltpu.async_copy(h.at[idxs.at[j]], rbuf[b], gsem[b])

            def gather_wait(j, b):
                pltpu.make_async_copy(h.at[idxs.at[j]], rbuf[b],
                                      gsem[b]).wait()

            def scatter_start(j, b):
                pltpu.async_copy(rbuf[b], acc.at[idxd.at[j]], ssem[b],
                                 add=True)

            def scatter_wait(j, b):
                pltpu.make_async_copy(rbuf[b], acc.at[idxd.at[j]],
                                      ssem[b]).wait()

            def group(g, carry):
                gr = row0 + g * grp

                @pl.when(g > 0)
                def _():
                    pltpu.async_copy(src_r.at[pl.ds(gr, grp)], idxs, sem4)
                    pltpu.async_copy(dst_r.at[pl.ds(gr, grp)], idxd, sem4)
                    pltpu.make_async_copy(src_r.at[pl.ds(gr, grp)], idxs,
                                          sem4).wait()
                    pltpu.make_async_copy(dst_r.at[pl.ds(gr, grp)], idxd,
                                          sem4).wait()
                gather_start(0, 0)
                for j in range(grp):
                    b = j % 2
                    if j + 1 < grp:
                        if j >= 1:
                            scatter_wait(j - 1, 1 - b)
                        gather_start(j + 1, 1 - b)
                    gather_wait(j, b)
                    scatter_start(j, b)
                    if j >= grp - 2:
                        scatter_wait(j, b)
                return carry
            lax.fori_loop(0, ngrp, group, 0)

        @pl.when(c == 0)
        def _():
            chunk(h_a)

        @pl.when(c == 1)
        def _():
            chunk(h_b)

        plsc.subcore_barrier()

        def copy_out(out):
            @pl.when(s < _NS - 1)
            def _():
                pltpu.sync_copy(acc.at[pl.ds(s * _RPT, _RPT)],
                                out.at[pl.ds(s * _RPT, _RPT)])

            @pl.when(s == _NS - 1)
            def _():
                pltpu.sync_copy(acc.at[pl.ds((_NS - 1) * _RPT, _RPT_LAST)],
                                out.at[pl.ds((_NS - 1) * _RPT, _RPT_LAST)])

        @pl.when(c == 0)
        def _():
            copy_out(out_a)

        @pl.when(c == 1)
        def _():
            copy_out(out_b)

    mesh = plsc.VectorSubcoreMesh(core_axis_name="c", subcore_axis_name="s")
    return pl.kernel(
        body,
        out_type=(jax.ShapeDtypeStruct((_N, 128), jnp.float32),
                  jax.ShapeDtypeStruct((_N, 128), jnp.float32)),
        mesh=mesh,
        scratch_types=[
            pltpu.VMEM_SHARED((_N, 128), jnp.float32),
            pltpu.VMEM((grp, _CH), jnp.int32),
            pltpu.VMEM((grp, _CH), jnp.int32),
            pltpu.VMEM((_CH, 128), jnp.float32),
            pltpu.VMEM((_CH, 128), jnp.float32),
            pltpu.SemaphoreType.DMA,
            pltpu.SemaphoreType.DMA,
            pltpu.SemaphoreType.DMA,
            pltpu.SemaphoreType.DMA,
            pltpu.SemaphoreType.DMA,
            pltpu.SemaphoreType.DMA,
        ],
    )


def _row_block(din):
    return pl.BlockSpec((_BR, din), lambda p, i: (i, 0))


def _agg_block():
    return pl.BlockSpec((_BR, 128), lambda p, i: (i * (1 - p), 0))


def _full_block(shape):
    return pl.BlockSpec(shape, lambda p, i: tuple(0 for _ in shape))


def _layer_common(i, zp, zscr, s1_ref):
    zscr[pl.ds(i * _BR, _BR), :] = zp
    ss = jnp.concatenate(
        [jnp.sum(zp, axis=0, keepdims=True),
         jnp.sum(zp * zp, axis=0, keepdims=True),
         jnp.zeros((6, zp.shape[1]), jnp.float32)], axis=0)

    @pl.when(i == 0)
    def _():
        s1_ref[...] = ss

    @pl.when(i != 0)
    def _():
        s1_ref[...] = s1_ref[...] + ss


def _layer_finalize(gam_ref, bet_ref, s1_ref, coef_ref):
    inv_n = 1.0 / _N
    mean = s1_ref[0:1, :] * inv_n
    var = s1_ref[1:2, :] * inv_n - mean * mean
    scale = gam_ref[...] / jnp.sqrt(var + 1e-5)
    shift = bet_ref[...] - mean * scale
    coef_ref[...] = jnp.concatenate(
        [scale, shift, jnp.zeros((6, scale.shape[1]), jnp.float32)], axis=0)


def _layer0_body(x_ref, p0_ref, p1_ref, w1_ref, b1_ref, w2_ref, b2_ref,
                 gam_ref, bet_ref, o0_ref, o1_ref,
                 zscr, s1_ref, coef_ref):
    p = pl.program_id(0)
    i = pl.program_id(1)

    @pl.when(p == 0)
    def _():
        zin = x_ref[...] + p0_ref[...] + p1_ref[...]
        z1 = jnp.maximum(jnp.dot(zin, w1_ref[...]) + b1_ref[...], 0.0)
        zp = jnp.dot(z1, w2_ref[...]) + b2_ref[...]
        _layer_common(i, zp, zscr, s1_ref)

    @pl.when(p == 1)
    def _():
        @pl.when(i == 0)
        def _():
            _layer_finalize(gam_ref, bet_ref, s1_ref, coef_ref)
        y = zscr[pl.ds(i * _BR, _BR), :] * coef_ref[0:1, :] \
            + coef_ref[1:2, :]
        y = jnp.maximum(y, 0.0)
        o0_ref[...] = y[:, 0:128]
        o1_ref[...] = y[:, 128:256]


def _layer12_body(h0_ref, h1_ref, a0_ref, a1_ref, w1_ref, b1_ref, w2_ref,
                  b2_ref, gam_ref, bet_ref, o0_ref, o1_ref,
                  zscr, s1_ref, coef_ref, hscr0, hscr1):
    p = pl.program_id(0)
    i = pl.program_id(1)

    @pl.when(p == 0)
    def _():
        h0b = h0_ref[...]
        h1b = h1_ref[...]
        hscr0[pl.ds(i * _BR, _BR), :] = h0b
        hscr1[pl.ds(i * _BR, _BR), :] = h1b
        zin0 = h0b + a0_ref[...]
        zin1 = h1b + a1_ref[...]
        z1 = jnp.maximum(
            jnp.dot(zin0, w1_ref[0:128, :])
            + jnp.dot(zin1, w1_ref[128:256, :]) + b1_ref[...], 0.0)
        zp = jnp.dot(z1, w2_ref[...]) + b2_ref[...]
        _layer_common(i, zp, zscr, s1_ref)

    @pl.when(p == 1)
    def _():
        @pl.when(i == 0)
        def _():
            _layer_finalize(gam_ref, bet_ref, s1_ref, coef_ref)
        y = zscr[pl.ds(i * _BR, _BR), :] * coef_ref[0:1, :] \
            + coef_ref[1:2, :]
        y = jnp.maximum(y, 0.0)
        o0_ref[...] = y[:, 0:128] + hscr0[pl.ds(i * _BR, _BR), :]
        o1_ref[...] = y[:, 128:256] + hscr1[pl.ds(i * _BR, _BR), :]


def _layer2_pool_body(h0_ref, h1_ref, a0_ref, a1_ref, w1_ref, b1_ref,
                      w2_ref, b2_ref, gam_ref, bet_ref, batch_ref,
                      wm1_ref, bm1_ref, wm2_ref, bm2_ref, out_ref,
                      zscr, s1_ref, coef_ref, sums, counts, hscr0, hscr1):
    p = pl.program_id(0)
    i = pl.program_id(1)

    @pl.when(p == 0)
    def _():
        h0b = h0_ref[...]
        h1b = h1_ref[...]
        hscr0[pl.ds(i * _BR, _BR), :] = h0b
        hscr1[pl.ds(i * _BR, _BR), :] = h1b
        zin0 = h0b + a0_ref[...]
        zin1 = h1b + a1_ref[...]
        z1 = jnp.maximum(
            jnp.dot(zin0, w1_ref[0:128, :])
            + jnp.dot(zin1, w1_ref[128:256, :]) + b1_ref[...], 0.0)
        zp = jnp.dot(z1, w2_ref[...]) + b2_ref[...]
        _layer_common(i, zp, zscr, s1_ref)

    @pl.when(p == 1)
    def _():
        @pl.when(i == 0)
        def _():
            _layer_finalize(gam_ref, bet_ref, s1_ref, coef_ref)
        y = zscr[pl.ds(i * _BR, _BR), :] * coef_ref[0:1, :] \
            + coef_ref[1:2, :]
        y = jnp.maximum(y, 0.0)
        y0 = y[:, 0:128] + hscr0[pl.ds(i * _BR, _BR), :]
        y1 = y[:, 128:256] + hscr1[pl.ds(i * _BR, _BR), :]
        b = batch_ref[0]
        gids = lax.broadcasted_iota(jnp.int32, (_G, _BR), 0)
        mask = (gids == b).astype(jnp.float32)
        hcat = jnp.concatenate([y0, y1], axis=1)
        part = jnp.dot(mask, hcat)
        cnt = jnp.broadcast_to(jnp.sum(mask, axis=1, keepdims=True),
                               (_G, 128))

        @pl.when(i == 0)
        def _():
            sums[...] = part
            counts[...] = cnt

        @pl.when(i != 0)
        def _():
            sums[...] = sums[...] + part
            counts[...] = counts[...] + cnt

        @pl.when(i == _NSTEPS - 1)
        def _():
            hg = sums[...] / jnp.maximum(counts[:, 0:1], 1.0)
            t = jnp.maximum(jnp.dot(hg, wm1_ref[...]) + bm1_ref[...], 0.0)
            out_ref[...] = jnp.dot(t, wm2_ref[...]) + bm2_ref[...]


_LAYER_SCRATCH = [
    pltpu.VMEM((_N, 256), jnp.float32),
    pltpu.VMEM((8, 256), jnp.float32),
    pltpu.VMEM((8, 256), jnp.float32),
]

_HRES_SCRATCH = [
    pltpu.VMEM((_N, 128), jnp.float32),
    pltpu.VMEM((_N, 128), jnp.float32),
]


def _tc_layer2_pool(h0, h1, a0, a1, w1, b1, w2, b2, gam, bet, batch3,
                    wm1, bm1, wm2p, bm2p):
    return pl.pallas_call(
        _layer2_pool_body,
        grid=(2, _NSTEPS),
        in_specs=[_agg_block(), _agg_block(), _agg_block(),
                  _agg_block(),
                  _full_block((256, 256)), _full_block((1, 256)),
                  _full_block((256, 256)), _full_block((1, 256)),
                  _full_block((1, 256)), _full_block((1, 256)),
                  pl.BlockSpec((1, 1, _BR), lambda p, i: (i * p, 0, 0)),
                  _full_block((256, 256)), _full_block((1, 256)),
                  _full_block((256, 128)), _full_block((1, 128))],
        out_specs=_full_block((_G, 128)),
        out_shape=jax.ShapeDtypeStruct((_G, 128), jnp.float32),
        scratch_shapes=_LAYER_SCRATCH + [
            pltpu.VMEM((_G, 256), jnp.float32),
            pltpu.VMEM((_G, 128), jnp.float32)] + _HRES_SCRATCH,
    )(h0, h1, a0, a1, w1, b1, w2, b2, gam, bet, batch3,
      wm1, bm1, wm2p, bm2p)

_LAYER_OUT = dict(
    out_specs=[pl.BlockSpec((_BR, 128), lambda p, i: (i, 0)),
               pl.BlockSpec((_BR, 128), lambda p, i: (i, 0))],
    out_shape=[jax.ShapeDtypeStruct((_N, 128), jnp.float32),
               jax.ShapeDtypeStruct((_N, 128), jnp.float32)],
    scratch_shapes=_LAYER_SCRATCH,
)


def _tc_layer0(x, p0, p1, w1, b1, w2, b2, gam, bet):
    return pl.pallas_call(
        _layer0_body,
        grid=(2, _NSTEPS),
        in_specs=[_row_block(128), _agg_block(), _agg_block(),
                  _full_block((128, 256)), _full_block((1, 256)),
                  _full_block((256, 256)), _full_block((1, 256)),
                  _full_block((1, 256)), _full_block((1, 256))],
        **_LAYER_OUT,
    )(x, p0, p1, w1, b1, w2, b2, gam, bet)


def _tc_layer12(h0, h1, a0, a1, w1, b1, w2, b2, gam, bet):
    return pl.pallas_call(
        _layer12_body,
        grid=(2, _NSTEPS),
        in_specs=[_agg_block(), _agg_block(), _agg_block(),
                  _agg_block(),
                  _full_block((256, 256)), _full_block((1, 256)),
                  _full_block((256, 256)), _full_block((1, 256)),
                  _full_block((1, 256)), _full_block((1, 256))],
        out_specs=_LAYER_OUT["out_specs"],
        out_shape=_LAYER_OUT["out_shape"],
        scratch_shapes=_LAYER_SCRATCH + _HRES_SCRATCH,
    )(h0, h1, a0, a1, w1, b1, w2, b2, gam, bet)


def kernel(x, edge_index, batch, W1_0, b1_0, W2_0, b2_0, gam_0, bet_0,
           W1_1, b1_1, W2_1, b2_1, gam_1, bet_1,
           W1_2, b1_2, W2_2, b2_2, gam_2, bet_2, Wm1, bm1, Wm2, bm2):
    src_r = edge_index[0].astype(jnp.int32).reshape(_E // _CH, _CH)
    dst_r = edge_index[1].astype(jnp.int32).reshape(_E // _CH, _CH)
    zblk = jnp.zeros((_RPT_LAST, 128), jnp.float32)

    r = lambda v: v.reshape(1, -1)
    agg_edges = _make_sc_agg(split_edges=True)
    agg_feats = _make_sc_agg(split_edges=False)

    p0, p1 = agg_edges(x, x, src_r, dst_r, zblk)
    h0, h1 = _tc_layer0(x, p0, p1, W1_0, r(b1_0), W2_0, r(b2_0),
                        r(gam_0), r(bet_0))

    a0, a1 = agg_feats(h0, h1, src_r, dst_r, zblk)
    h0, h1 = _tc_layer12(h0, h1, a0, a1, W1_1, r(b1_1), W2_1, r(b2_1),
                         r(gam_1), r(bet_1))

    batch3 = batch.astype(jnp.int32).reshape(_NSTEPS, 1, _BR)
    wm2p = jnp.pad(Wm2, ((0, 0), (0, 118)))
    bm2p = jnp.pad(bm2, (0, 118)).reshape(1, 128)
    a0, a1 = agg_feats(h0, h1, src_r, dst_r, zblk)
    out = _tc_layer2_pool(h0, h1, a0, a1, W1_2, r(b1_2), W2_2, r(b2_2),
                          r(gam_2), r(bet_2), batch3,
                          Wm1, r(bm1), wm2p, bm2p)
    return out[:, :10]


# speedup vs baseline: 9.4515x; 1.0012x over previous
import jax
import jax.numpy as jnp
from jax import lax
from jax.experimental import pallas as pl
from jax.experimental.pallas import tpu as pltpu
from jax.experimental.pallas import tpu_sc as plsc

_N = 10000
_E = 320000
_G = 64
_NC = 2
_NS = 16
_CH = 125
_RPT = 624
_RPT_LAST = _N - (_NS - 1) * _RPT
_GRP = 16

_BR = 2000
_NSTEPS = _N // _BR


def _make_sc_agg(split_edges: bool):
    if split_edges:
        nch = (_E // _NC) // _NS // _CH
        core_row_off = (_E // _NC) // _CH
        grp = 40
    else:
        nch = _E // _NS // _CH
        core_row_off = 0
        grp = 2 * _GRP
    ngrp = nch // grp

    def body(h_a, h_b, src_r, dst_r, zblk, out_a, out_b,
             acc, idxs, idxd, rows0, rows1,
             sem0, sem1, sem2, sem3, sem4, sem5):
        c = lax.axis_index("c")
        s = lax.axis_index("s")

        @pl.when(s < _NS - 1)
        def _():
            pltpu.async_copy(zblk.at[pl.ds(0, _RPT)],
                             acc.at[pl.ds(s * _RPT, _RPT)], sem5)

        @pl.when(s == _NS - 1)
        def _():
            pltpu.async_copy(zblk, acc.at[pl.ds((_NS - 1) * _RPT, _RPT_LAST)],
                             sem5)

        row0 = c * core_row_off + s * nch
        pltpu.async_copy(src_r.at[pl.ds(row0, grp)], idxs, sem4)
        pltpu.async_copy(dst_r.at[pl.ds(row0, grp)], idxd, sem4)
        pltpu.make_async_copy(src_r.at[pl.ds(row0, grp)], idxs, sem4).wait()
        pltpu.make_async_copy(dst_r.at[pl.ds(row0, grp)], idxd, sem4).wait()

        @pl.when(s < _NS - 1)
        def _():
            pltpu.make_async_copy(zblk.at[pl.ds(0, _RPT)],
                                  acc.at[pl.ds(s * _RPT, _RPT)], sem5).wait()

        @pl.when(s == _NS - 1)
        def _():
            pltpu.make_async_copy(zblk,
                                  acc.at[pl.ds((_NS - 1) * _RPT, _RPT_LAST)],
                                  sem5).wait()

        plsc.subcore_barrier()

        def chunk(h):
            rbuf = (rows0, rows1)
            gsem = (sem0, sem1)
            ssem = (sem2, sem3)

            def gather_start(j, b):
                pltpu.async_copy(h.at[idxs.at[j]], rbuf[b], gsem[b])

            def gather_wait(j, b):
                pltpu.make_async_copy(h.at[idxs.at[j]], rbuf[b],
                                      gsem[b]).wait()

            def scatter_start(j, b):
                pltpu.async_copy(rbuf[b], acc.at[idxd.at[j]], ssem[b],
                                 add=True)

            def scatter_wait(j, b):
                pltpu.make_async_copy(rbuf[b], acc.at[idxd.at[j]],
                                      ssem[b]).wait()

            def group(g, carry):
                gr = row0 + g * grp

                @pl.when(g > 0)
                def _():
                    pltpu.async_copy(src_r.at[pl.ds(gr, grp)], idxs, sem4)
                    pltpu.async_copy(dst_r.at[pl.ds(gr, grp)], idxd, sem4)
                    pltpu.make_async_copy(src_r.at[pl.ds(gr, grp)], idxs,
                                          sem4).wait()
                    pltpu.make_async_copy(dst_r.at[pl.ds(gr, grp)], idxd,
                                          sem4).wait()
                gather_start(0, 0)
                for j in range(grp):
                    b = j % 2
                    if j + 1 < grp:
                        if j >= 1:
                            scatter_wait(j - 1, 1 - b)
                        gather_start(j + 1, 1 - b)
                    gather_wait(j, b)
                    scatter_start(j, b)
                    if j >= grp - 2:
                        scatter_wait(j, b)
                return carry
            lax.fori_loop(0, ngrp, group, 0)

        @pl.when(c == 0)
        def _():
            chunk(h_a)

        @pl.when(c == 1)
        def _():
            chunk(h_b)

        plsc.subcore_barrier()

        def copy_out(out):
            @pl.when(s < _NS - 1)
            def _():
                pltpu.sync_copy(acc.at[pl.ds(s * _RPT, _RPT)],
                                out.at[pl.ds(s * _RPT, _RPT)])

            @pl.when(s == _NS - 1)
            def _():
                pltpu.sync_copy(acc.at[pl.ds((_NS - 1) * _RPT, _RPT_LAST)],
                                out.at[pl.ds((_NS - 1) * _RPT, _RPT_LAST)])

        @pl.when(c == 0)
        def _():
            copy_out(out_a)

        @pl.when(c == 1)
        def _():
            copy_out(out_b)

    mesh = plsc.VectorSubcoreMesh(core_axis_name="c", subcore_axis_name="s")
    return pl.kernel(
        body,
        out_type=(jax.ShapeDtypeStruct((_N, 128), jnp.float32),
                  jax.ShapeDtypeStruct((_N, 128), jnp.float32)),
        mesh=mesh,
        scratch_types=[
            pltpu.VMEM_SHARED((_N, 128), jnp.float32),
            pltpu.VMEM((grp, _CH), jnp.int32),
            pltpu.VMEM((grp, _CH), jnp.int32),
            pltpu.VMEM((_CH, 128), jnp.float32),
            pltpu.VMEM((_CH, 128), jnp.float32),
            pltpu.SemaphoreType.DMA,
            pltpu.SemaphoreType.DMA,
            pltpu.SemaphoreType.DMA,
            pltpu.SemaphoreType.DMA,
            pltpu.SemaphoreType.DMA,
            pltpu.SemaphoreType.DMA,
        ],
    )


def _row_block(din):
    return pl.BlockSpec((_BR, din), lambda p, i: (i, 0))


def _agg_block():
    return pl.BlockSpec((_BR, 128), lambda p, i: (i * (1 - p), 0))


def _full_block(shape):
    return pl.BlockSpec(shape, lambda p, i: tuple(0 for _ in shape))


def _layer_common(i, zp, zscr, s1_ref):
    zscr[pl.ds(i * _BR, _BR), :] = zp
    ss = jnp.concatenate(
        [jnp.sum(zp, axis=0, keepdims=True),
         jnp.sum(zp * zp, axis=0, keepdims=True),
         jnp.zeros((6, zp.shape[1]), jnp.float32)], axis=0)

    @pl.when(i == 0)
    def _():
        s1_ref[...] = ss

    @pl.when(i != 0)
    def _():
        s1_ref[...] = s1_ref[...] + ss


def _layer_finalize(gam_ref, bet_ref, s1_ref, coef_ref):
    inv_n = 1.0 / _N
    mean = s1_ref[0:1, :] * inv_n
    var = s1_ref[1:2, :] * inv_n - mean * mean
    scale = gam_ref[...] / jnp.sqrt(var + 1e-5)
    shift = bet_ref[...] - mean * scale
    coef_ref[...] = jnp.concatenate(
        [scale, shift, jnp.zeros((6, scale.shape[1]), jnp.float32)], axis=0)


def _layer0_body(x_ref, p0_ref, p1_ref, w1_ref, b1_ref, w2_ref, b2_ref,
                 gam_ref, bet_ref, o0_ref, o1_ref,
                 zscr, s1_ref, coef_ref):
    p = pl.program_id(0)
    i = pl.program_id(1)

    @pl.when(p == 0)
    def _():
        zin = x_ref[...] + p0_ref[...] + p1_ref[...]
        z1 = jnp.maximum(jnp.dot(zin, w1_ref[...]) + b1_ref[...], 0.0)
        zp = jnp.dot(z1, w2_ref[...]) + b2_ref[...]
        _layer_common(i, zp, zscr, s1_ref)

    @pl.when(p == 1)
    def _():
        @pl.when(i == 0)
        def _():
            _layer_finalize(gam_ref, bet_ref, s1_ref, coef_ref)
        y = zscr[pl.ds(i * _BR, _BR), :] * coef_ref[0:1, :] \
            + coef_ref[1:2, :]
        y = jnp.maximum(y, 0.0)
        o0_ref[...] = y[:, 0:128]
        o1_ref[...] = y[:, 128:256]


def _layer12_body(h0_ref, h1_ref, a0_ref, a1_ref, w1_ref, b1_ref, w2_ref,
                  b2_ref, gam_ref, bet_ref, o0_ref, o1_ref,
                  zscr, s1_ref, coef_ref, hscr0, hscr1):
    p = pl.program_id(0)
    i = pl.program_id(1)

    @pl.when(p == 0)
    def _():
        h0b = h0_ref[...]
        h1b = h1_ref[...]
        hscr0[pl.ds(i * _BR, _BR), :] = h0b
        hscr1[pl.ds(i * _BR, _BR), :] = h1b
        zin0 = h0b + a0_ref[...]
        zin1 = h1b + a1_ref[...]
        z1 = jnp.maximum(
            jnp.dot(zin0, w1_ref[0:128, :])
            + jnp.dot(zin1, w1_ref[128:256, :]) + b1_ref[...], 0.0)
        zp = jnp.dot(z1, w2_ref[...]) + b2_ref[...]
        _layer_common(i, zp, zscr, s1_ref)

    @pl.when(p == 1)
    def _():
        @pl.when(i == 0)
        def _():
            _layer_finalize(gam_ref, bet_ref, s1_ref, coef_ref)
        y = zscr[pl.ds(i * _BR, _BR), :] * coef_ref[0:1, :] \
            + coef_ref[1:2, :]
        y = jnp.maximum(y, 0.0)
        o0_ref[...] = y[:, 0:128] + hscr0[pl.ds(i * _BR, _BR), :]
        o1_ref[...] = y[:, 128:256] + hscr1[pl.ds(i * _BR, _BR), :]


def _layer2_pool_body(h0_ref, h1_ref, a0_ref, a1_ref, w1_ref, b1_ref,
                      w2_ref, b2_ref, gam_ref, bet_ref, batch_ref,
                      wm1_ref, bm1_ref, wm2_ref, bm2_ref, out_ref,
                      zscr, s1_ref, coef_ref, sums, counts, hscr0, hscr1):
    p = pl.program_id(0)
    i = pl.program_id(1)

    @pl.when(p == 0)
    def _():
        h0b = h0_ref[...]
        h1b = h1_ref[...]
        hscr0[pl.ds(i * _BR, _BR), :] = h0b
        hscr1[pl.ds(i * _BR, _BR), :] = h1b
        zin0 = h0b + a0_ref[...]
        zin1 = h1b + a1_ref[...]
        z1 = jnp.maximum(
            jnp.dot(zin0, w1_ref[0:128, :])
            + jnp.dot(zin1, w1_ref[128:256, :]) + b1_ref[...], 0.0)
        zp = jnp.dot(z1, w2_ref[...]) + b2_ref[...]
        _layer_common(i, zp, zscr, s1_ref)

    @pl.when(p == 1)
    def _():
        @pl.when(i == 0)
        def _():
            _layer_finalize(gam_ref, bet_ref, s1_ref, coef_ref)
        y = zscr[pl.ds(i * _BR, _BR), :] * coef_ref[0:1, :] \
            + coef_ref[1:2, :]
        y = jnp.maximum(y, 0.0)
        y0 = y[:, 0:128] + hscr0[pl.ds(i * _BR, _BR), :]
        y1 = y[:, 128:256] + hscr1[pl.ds(i * _BR, _BR), :]
        b = batch_ref[0]
        gids = lax.broadcasted_iota(jnp.int32, (_G, _BR), 0)
        mask = (gids == b).astype(jnp.float32)
        hcat = jnp.concatenate([y0, y1], axis=1)
        part = jnp.dot(mask, hcat)
        cnt = jnp.broadcast_to(jnp.sum(mask, axis=1, keepdims=True),
                               (_G, 128))

        @pl.when(i == 0)
        def _():
            sums[...] = part
            counts[...] = cnt

        @pl.when(i != 0)
        def _():
            sums[...] = sums[...] + part
            counts[...] = counts[...] + cnt

        @pl.when(i == _NSTEPS - 1)
        def _():
            hg = sums[...] / jnp.maximum(counts[:, 0:1], 1.0)
            t = jnp.maximum(jnp.dot(hg, wm1_ref[...]) + bm1_ref[...], 0.0)
            out_ref[...] = jnp.dot(t, wm2_ref[...]) + bm2_ref[...]


_LAYER_SCRATCH = [
    pltpu.VMEM((_N, 256), jnp.float32),
    pltpu.VMEM((8, 256), jnp.float32),
    pltpu.VMEM((8, 256), jnp.float32),
]

_HRES_SCRATCH = [
    pltpu.VMEM((_N, 128), jnp.float32),
    pltpu.VMEM((_N, 128), jnp.float32),
]


def _tc_layer2_pool(h0, h1, a0, a1, w1, b1, w2, b2, gam, bet, batch3,
                    wm1, bm1, wm2p, bm2p):
    return pl.pallas_call(
        _layer2_pool_body,
        grid=(2, _NSTEPS),
        in_specs=[_agg_block(), _agg_block(), _agg_block(),
                  _agg_block(),
                  _full_block((256, 256)), _full_block((1, 256)),
                  _full_block((256, 256)), _full_block((1, 256)),
                  _full_block((1, 256)), _full_block((1, 256)),
                  pl.BlockSpec((1, 1, _BR), lambda p, i: (i * p, 0, 0)),
                  _full_block((256, 256)), _full_block((1, 256)),
                  _full_block((256, 128)), _full_block((1, 128))],
        out_specs=_full_block((_G, 128)),
        out_shape=jax.ShapeDtypeStruct((_G, 128), jnp.float32),
        scratch_shapes=_LAYER_SCRATCH + [
            pltpu.VMEM((_G, 256), jnp.float32),
            pltpu.VMEM((_G, 128), jnp.float32)] + _HRES_SCRATCH,
    )(h0, h1, a0, a1, w1, b1, w2, b2, gam, bet, batch3,
      wm1, bm1, wm2p, bm2p)

_LAYER_OUT = dict(
    out_specs=[pl.BlockSpec((_BR, 128), lambda p, i: (i, 0)),
               pl.BlockSpec((_BR, 128), lambda p, i: (i, 0))],
    out_shape=[jax.ShapeDtypeStruct((_N, 128), jnp.float32),
               jax.ShapeDtypeStruct((_N, 128), jnp.float32)],
    scratch_shapes=_LAYER_SCRATCH,
)


def _tc_layer0(x, p0, p1, w1, b1, w2, b2, gam, bet):
    return pl.pallas_call(
        _layer0_body,
        grid=(2, _NSTEPS),
        in_specs=[_row_block(128), _agg_block(), _agg_block(),
                  _full_block((128, 256)), _full_block((1, 256)),
                  _full_block((256, 256)), _full_block((1, 256)),
                  _full_block((1, 256)), _full_block((1, 256))],
        **_LAYER_OUT,
    )(x, p0, p1, w1, b1, w2, b2, gam, bet)


def _tc_layer12(h0, h1, a0, a1, w1, b1, w2, b2, gam, bet):
    return pl.pallas_call(
        _layer12_body,
        grid=(2, _NSTEPS),
        in_specs=[_agg_block(), _agg_block(), _agg_block(),
                  _agg_block(),
                  _full_block((256, 256)), _full_block((1, 256)),
                  _full_block((256, 256)), _full_block((1, 256)),
                  _full_block((1, 256)), _full_block((1, 256))],
        out_specs=_LAYER_OUT["out_specs"],
        out_shape=_LAYER_OUT["out_shape"],
        scratch_shapes=_LAYER_SCRATCH + _HRES_SCRATCH,
    )(h0, h1, a0, a1, w1, b1, w2, b2, gam, bet)


def kernel(x, edge_index, batch, W1_0, b1_0, W2_0, b2_0, gam_0, bet_0,
           W1_1, b1_1, W2_1, b2_1, gam_1, bet_1,
           W1_2, b1_2, W2_2, b2_2, gam_2, bet_2, Wm1, bm1, Wm2, bm2):
    src_r = edge_index[0].astype(jnp.int32).reshape(_E // _CH, _CH)
    dst_r = edge_index[1].astype(jnp.int32).reshape(_E // _CH, _CH)
    zblk = jnp.zeros((_RPT_LAST, 128), jnp.float32)

    r = lambda v: v.reshape(1, -1)
    agg_edges = _make_sc_agg(split_edges=True)
    agg_feats = _make_sc_agg(split_edges=False)

    p0, p1 = agg_edges(x, x, src_r, dst_r, zblk)
    h0, h1 = _tc_layer0(x, p0, p1, W1_0, r(b1_0), W2_0, r(b2_0),
                        r(gam_0), r(bet_0))

    a0, a1 = agg_feats(h0, h1, src_r, dst_r, zblk)
    h0, h1 = _tc_layer12(h0, h1, a0, a1, W1_1, r(b1_1), W2_1, r(b2_1),
                         r(gam_1), r(bet_1))

    batch3 = batch.astype(jnp.int32).reshape(_NSTEPS, 1, _BR)
    wm2p = jnp.pad(Wm2, ((0, 0), (0, 118)))
    bm2p = jnp.pad(bm2, (0, 118)).reshape(1, 128)
    a0, a1 = agg_feats(h0, h1, src_r, dst_r, zblk)
    out = _tc_layer2_pool(h0, h1, a0, a1, W1_2, r(b1_2), W2_2, r(b2_2),
                          r(gam_2), r(bet_2), batch3,
                          Wm1, r(bm1), wm2p, bm2p)
    return out[:, :10]
